# baseline (speedup 1.0000x reference)
import functools

import jax
import jax.numpy as jnp
import numpy as np
from jax import lax
from jax.experimental import pallas as pl
from jax.experimental.pallas import tpu as pltpu
from jax.experimental.pallas import tpu_sc as plsc

_INTERPRET = False


def _sc_gather(table, idx):
    G = idx.shape[0]
    D = table.shape[1]
    info = plsc.get_sparse_core_info()
    nc, ns = info.num_cores, info.num_subcores
    nw = nc * ns
    assert G % nw == 0, (G, nw)
    per_w = G // nw
    chunk = 8
    while chunk * 2 <= per_w and chunk * 2 <= 128 and (chunk * 2) * (D + 1) <= 110000:
        chunk *= 2
    chunk = min(chunk, per_w)
    nch = per_w // chunk
    mesh = plsc.VectorSubcoreMesh(core_axis_name="c", subcore_axis_name="s")

    @functools.partial(
        pl.kernel,
        mesh=mesh,
        out_type=jax.ShapeDtypeStruct((G, D), jnp.float32),
        scratch_types=[
            pltpu.VMEM((chunk,), jnp.int32),
            pltpu.VMEM((chunk, D), jnp.float32),
            pltpu.SemaphoreType.DMA,
        ],
    )
    def gk(tab_hbm, idx_hbm, out_hbm, idx_v, rows_v, sem):
        wid = lax.axis_index("s") * nc + lax.axis_index("c")
        base = wid * per_w

        def one(ci):
            off = base + ci * chunk
            pltpu.sync_copy(idx_hbm.at[pl.ds(off, chunk)], idx_v)
            pltpu.async_copy(tab_hbm.at[idx_v], rows_v, sem).wait()
            pltpu.sync_copy(rows_v, out_hbm.at[pl.ds(off, chunk)])

        if nch == 1:
            one(0)
        else:
            def body(ci, carry):
                one(ci)
                return carry

            lax.fori_loop(0, nch, body, 0)

    return gk(table, idx)


_gather = _sc_gather


def _mlp_in_body(x_ref, w1, b1, w2, b2, o_ref):
    h = jnp.maximum(x_ref[...] @ w1[...] + b1[...], 0.0)
    o_ref[...] = h @ w2[...] + b2[...]


def _prep_body(f_ref, xz_ref, fc1w, fc1b, wq, q_ref, tab_ref):
    xx = f_ref[...] @ fc1w[...] + fc1b[...]
    q_ref[...] = xx @ wq[...]
    tab_ref[:, 0:256] = xx
    tab_ref[:, 256:272] = xz_ref[...]
    tab_ref[:, 272:384] = jnp.zeros_like(tab_ref[:, 272:384])


def _knn_body(K, N, q_ref, p_ref, o_ref):
    q = q_ref[0]
    p = p_ref[0]
    qq = jnp.sum(q * q, axis=1, keepdims=True)
    pp = jnp.sum(p * p, axis=0, keepdims=True)
    d = qq + pp - 2.0 * (q @ p)
    iota = lax.broadcasted_iota(jnp.int32, d.shape, 1)
    cols = []
    for _ in range(K):
        m = jnp.min(d, axis=1, keepdims=True)
        sel = jnp.min(jnp.where(d == m, iota, N), axis=1, keepdims=True)
        cols.append(sel)
        d = jnp.where(iota == sel, jnp.float32(np.inf), d)
    o_ref[0] = jnp.concatenate(cols, axis=1)


def _fps_body(B, M, N, xr_ref, xt_ref, o_ref, dist_ref):
    iota = lax.broadcasted_iota(jnp.int32, (1, N), 1)
    dist_ref[...] = jnp.full((B, N), 1e10, jnp.float32)

    def body(i, fars):
        out = []
        for b in range(B):
            far = fars[b]
            o_ref[b, pl.ds(i, 1), :] = xr_ref[b, pl.ds(far, 1), :]
            cx = xr_ref[b, far, 0]
            cy = xr_ref[b, far, 1]
            cz = xr_ref[b, far, 2]
            s0 = xt_ref[b, 0:1, :] - cx
            s1 = xt_ref[b, 1:2, :] - cy
            s2 = xt_ref[b, 2:3, :] - cz
            d2 = s0 * s0 + s1 * s1
            d2 = d2 + s2 * s2
            dist = jnp.minimum(dist_ref[b:b + 1, :], d2)
            dist_ref[b:b + 1, :] = dist
            m = jnp.max(dist)
            out.append(
                jnp.min(jnp.where(dist == m, iota, N)).astype(jnp.int32))
        return tuple(out)

    lax.fori_loop(0, M, body, tuple(jnp.int32(0) for _ in range(B)))


def _attn_body(K, g_ref, q_ref, xz_ref, pre_ref, wk, wv, fdw1, fdb1, fdw2,
               fdb2, fgw1, fgb1, fgw2, fgb2, fc2w, fc2b, o_ref):
    q = q_ref[0]
    xz = xz_ref[0]
    attn = []
    vp = []
    for j in range(K):
        xj = g_ref[0, j, :, 0:256]
        nx = g_ref[0, j, :, 256:272]
        kj = xj @ wk[...]
        vj = xj @ wv[...]
        rel = xz - nx
        h = jnp.maximum(rel @ fdw1[...] + fdb1[...], 0.0)
        pos = h @ fdw2[...] + fdb2[...]
        gj = jnp.maximum((q - kj + pos) @ fgw1[...] + fgb1[...], 0.0)
        aj = (gj @ fgw2[...] + fgb2[...]) * jnp.float32(1.0 / 16.0)
        attn.append(aj)
        vp.append(vj + pos)
    m = attn[0]
    for j in range(1, K):
        m = jnp.maximum(m, attn[j])
    num = None
    den = None
    for j in range(K):
        e = jnp.exp(attn[j] - m)
        den = e if den is None else den + e
        c = e * vp[j]
        num = c if num is None else num + c
    res = num / den
    o_ref[0] = res @ fc2w[...] + fc2b[...] + pre_ref[0]


def _td_body(K, d, g_ref, nx_ref, w1a, w1b, b1, g1, be1, w2, b2, g2, be2,
             o_ref):
    sq = np.float32(np.sqrt(1.0 + 1e-05))
    nx = nx_ref[0]
    acc = None
    for j in range(K):
        gp = g_ref[0, j, :, 0:d]
        gx = g_ref[0, j, :, d:d + 16]
        norm = gx - nx
        y = norm @ w1a[...] + gp @ w1b[...] + b1[...]
        y = jnp.maximum((y / sq) * g1[...] + be1[...], 0.0)
        y = y @ w2[...] + b2[...]
        y = jnp.maximum((y / sq) * g2[...] + be2[...], 0.0)
        acc = y if acc is None else jnp.maximum(acc, y)
    o_ref[0] = acc


def _full(shape):
    nd = len(shape)
    return pl.BlockSpec(shape, lambda *_: (0,) * nd)


def _input_mlp(xf, w1, b1, w2, b2):
    R = xf.shape[0]
    t = min(R, 1024)
    return pl.pallas_call(
        _mlp_in_body,
        grid=(R // t,),
        in_specs=[
            pl.BlockSpec((t, 3), lambda i: (i, 0)),
            _full(w1.shape), _full(b1.shape), _full(w2.shape), _full(b2.shape),
        ],
        out_specs=pl.BlockSpec((t, 32), lambda i: (i, 0)),
        out_shape=jax.ShapeDtypeStruct((R, 32), jnp.float32),
        interpret=_INTERPRET,
    )(xf, w1, b1, w2, b2)


def _prep(featsf, xz16f, p):
    R, d = featsf.shape
    t = min(R, 512)
    fc1b = p['fc1_b'].reshape(1, -1)
    q, tab = pl.pallas_call(
        _prep_body,
        grid=(R // t,),
        in_specs=[
            pl.BlockSpec((t, d), lambda i: (i, 0)),
            pl.BlockSpec((t, 16), lambda i: (i, 0)),
            _full((d, 256)), _full((1, 256)), _full((256, 256)),
        ],
        out_specs=[
            pl.BlockSpec((t, 256), lambda i: (i, 0)),
            pl.BlockSpec((t, 384), lambda i: (i, 0)),
        ],
        out_shape=[
            jax.ShapeDtypeStruct((R, 256), jnp.float32),
            jax.ShapeDtypeStruct((R, 384), jnp.float32),
        ],
        interpret=_INTERPRET,
    )(featsf, xz16f, p['fc1_w'], fc1b, p['wq'])
    return q, tab


def _knn(q16, pT16, K):
    B, M, _ = q16.shape
    N = pT16.shape[2]
    t = min(M, 256)
    return pl.pallas_call(
        functools.partial(_knn_body, K, N),
        grid=(B, M // t),
        in_specs=[
            pl.BlockSpec((1, t, 16), lambda b, i: (b, i, 0)),
            pl.BlockSpec((1, 16, N), lambda b, i: (b, 0, 0)),
        ],
        out_specs=pl.BlockSpec((1, t, K), lambda b, i: (b, i, 0)),
        out_shape=jax.ShapeDtypeStruct((B, M, K), jnp.int32),
        interpret=_INTERPRET,
    )(q16, pT16)


def _fps(xz16, xzT16, M):
    B, N, _ = xz16.shape
    return pl.pallas_call(
        functools.partial(_fps_body, B, M, N),
        grid=(1,),
        in_specs=[
            pl.BlockSpec((B, N, 16), lambda i: (0, 0, 0)),
            pl.BlockSpec((B, 16, N), lambda i: (0, 0, 0)),
        ],
        out_specs=pl.BlockSpec((B, M, 16), lambda i: (0, 0, 0)),
        out_shape=jax.ShapeDtypeStruct((B, M, 16), jnp.float32),
        scratch_shapes=[pltpu.VMEM((B, N), jnp.float32)],
        interpret=_INTERPRET,
    )(xz16, xzT16)


def _attn(gath, q, xz16, pre, p, K):
    B, M, d = pre.shape
    t = min(M, 128)
    fdw1 = jnp.concatenate(
        [p['fd_w1'], jnp.zeros((13, 256), jnp.float32)], axis=0)
    args = [gath, q, xz16, pre, p['wk'], p['wv'], fdw1,
            p['fd_b1'].reshape(1, -1), p['fd_w2'], p['fd_b2'].reshape(1, -1),
            p['fg_w1'], p['fg_b1'].reshape(1, -1),
            p['fg_w2'], p['fg_b2'].reshape(1, -1),
            p['fc2_w'], p['fc2_b'].reshape(1, -1)]
    return pl.pallas_call(
        functools.partial(_attn_body, K),
        grid=(B, M // t),
        in_specs=[
            pl.BlockSpec((1, K, t, 384), lambda b, i: (b, 0, i, 0)),
            pl.BlockSpec((1, t, 256), lambda b, i: (b, i, 0)),
            pl.BlockSpec((1, t, 16), lambda b, i: (b, i, 0)),
            pl.BlockSpec((1, t, d), lambda b, i: (b, i, 0)),
            _full((256, 256)), _full((256, 256)),
            _full((16, 256)), _full((1, 256)),
            _full((256, 256)), _full((1, 256)),
            _full((256, 256)), _full((1, 256)),
            _full((256, 256)), _full((1, 256)),
            _full((256, d)), _full((1, d)),
        ],
        out_specs=pl.BlockSpec((1, t, d), lambda b, i: (b, i, 0)),
        out_shape=jax.ShapeDtypeStruct((B, M, d), jnp.float32),
        interpret=_INTERPRET,
    )(*args)


def _td_conv(gath, nx16, p, d):
    B, K, M, D = gath.shape
    ch = p['convs'][0][0].shape[1]
    t = min(M, 128)
    (w1, b1, g1, be1), (w2, b2, g2, be2) = p['convs']
    w1a = jnp.concatenate(
        [w1[:3], jnp.zeros((13, ch), jnp.float32)], axis=0)
    w1b = w1[3:]
    args = [gath, nx16, w1a, w1b, b1.reshape(1, -1), g1.reshape(1, -1),
            be1.reshape(1, -1), w2, b2.reshape(1, -1), g2.reshape(1, -1),
            be2.reshape(1, -1)]
    return pl.pallas_call(
        functools.partial(_td_body, K, d),
        grid=(B, M // t),
        in_specs=[
            pl.BlockSpec((1, K, t, D), lambda b, i: (b, 0, i, 0)),
            pl.BlockSpec((1, t, 16), lambda b, i: (b, i, 0)),
            _full((16, ch)), _full((d, ch)), _full((1, ch)), _full((1, ch)),
            _full((1, ch)), _full((ch, ch)), _full((1, ch)), _full((1, ch)),
            _full((1, ch)),
        ],
        out_specs=pl.BlockSpec((1, t, ch), lambda b, i: (b, i, 0)),
        out_shape=jax.ShapeDtypeStruct((B, M, ch), jnp.float32),
        interpret=_INTERPRET,
    )(*args)


def _flat_idx(idx, N):
    B = idx.shape[0]
    off = (jnp.arange(B, dtype=jnp.int32) * N)[:, None, None]
    return (jnp.transpose(idx, (0, 2, 1)) + off).reshape(-1)


def _transformer(xz16, xzT16, feats, p, K):
    B, M, d = feats.shape
    idx = _knn(xz16, xzT16, K)
    q, tab = _prep(feats.reshape(B * M, d), xz16.reshape(B * M, 16), p)
    gath = _gather(tab, _flat_idx(idx, M)).reshape(B, K, M, 384)
    return _attn(gath, q.reshape(B, M, 256), xz16, feats, p, K)


def _transition(xz16, xzT16, points, M, p):
    B, N, d = points.shape
    new16 = _fps(xz16, xzT16, M)
    idx = _knn(new16, xzT16, 16)
    dpad = -(-(d + 16) // 128) * 128
    tab = jnp.concatenate(
        [points.reshape(B * N, d), xz16.reshape(B * N, 16),
         jnp.zeros((B * N, dpad - d - 16), jnp.float32)], axis=1)
    gath = _gather(tab, _flat_idx(idx, N)).reshape(B, 16, M, dpad)
    y = _td_conv(gath, new16, p, d)
    return new16, y


def kernel(x, params):
    B, N0, _ = x.shape
    xz16 = jnp.pad(x[..., :3], ((0, 0), (0, 0), (0, 13)))
    xzT16 = jnp.swapaxes(xz16, 1, 2)
    h = _input_mlp(x.reshape(B * N0, 3), params['in_w1'],
                   params['in_b1'].reshape(1, -1), params['in_w2'],
                   params['in_b2'].reshape(1, -1)).reshape(B, N0, 32)
    pts = _transformer(xz16, xzT16, h, params['t0'], 16)
    for i in range(4):
        M = N0 // 4 ** (i + 1)
        xz16, pts = _transition(xz16, xzT16, pts, M, params['td' + str(i)])
        xzT16 = jnp.swapaxes(xz16, 1, 2)
        pts = _transformer(xz16, xzT16, pts, params['t' + str(i + 1)],
                           min(16, M))
    return pts


# speedup vs baseline: 1.3066x; 1.3066x over previous
import functools

import jax
import jax.numpy as jnp
import numpy as np
from jax import lax
from jax.experimental import pallas as pl
from jax.experimental.pallas import tpu as pltpu
from jax.experimental.pallas import tpu_sc as plsc

_INTERPRET = False


def _sc_gather(table, idx):
    G = idx.shape[0]
    D = table.shape[1]
    info = plsc.get_sparse_core_info()
    nc, ns = info.num_cores, info.num_subcores
    nw = nc * ns
    assert G % nw == 0, (G, nw)
    per_w = G // nw
    chunk = 8
    while chunk * 2 <= per_w and chunk * 2 <= 128 and (chunk * 2) * (D + 1) <= 110000:
        chunk *= 2
    chunk = min(chunk, per_w)
    nch = per_w // chunk
    mesh = plsc.VectorSubcoreMesh(core_axis_name="c", subcore_axis_name="s")

    @functools.partial(
        pl.kernel,
        mesh=mesh,
        out_type=jax.ShapeDtypeStruct((G, D), jnp.float32),
        scratch_types=[
            pltpu.VMEM((chunk,), jnp.int32),
            pltpu.VMEM((chunk, D), jnp.float32),
            pltpu.SemaphoreType.DMA,
        ],
    )
    def gk(tab_hbm, idx_hbm, out_hbm, idx_v, rows_v, sem):
        wid = lax.axis_index("s") * nc + lax.axis_index("c")
        base = wid * per_w

        def one(ci):
            off = base + ci * chunk
            pltpu.sync_copy(idx_hbm.at[pl.ds(off, chunk)], idx_v)
            pltpu.async_copy(tab_hbm.at[idx_v], rows_v, sem).wait()
            pltpu.sync_copy(rows_v, out_hbm.at[pl.ds(off, chunk)])

        if nch == 1:
            one(0)
        else:
            def body(ci, carry):
                one(ci)
                return carry

            lax.fori_loop(0, nch, body, 0)

    return gk(table, idx)


_gather = _sc_gather


def _mlp_in_body(x_ref, w1, b1, w2, b2, o_ref):
    h = jnp.maximum(x_ref[...] @ w1[...] + b1[...], 0.0)
    o_ref[...] = h @ w2[...] + b2[...]


def _prep_body(f_ref, xz_ref, fc1w, fc1b, wq, wk, wv, q_ref, tab_ref):
    xx = f_ref[...] @ fc1w[...] + fc1b[...]
    q_ref[...] = xx @ wq[...]
    tab_ref[:, 0:256] = xx @ wk[...]
    tab_ref[:, 256:512] = xx @ wv[...]
    tab_ref[:, 512:528] = xz_ref[...]
    tab_ref[:, 528:640] = jnp.zeros_like(tab_ref[:, 528:640])


def _knn_body(K, N, q_ref, p_ref, o_ref):
    q = q_ref[0]
    p = p_ref[0]
    qq = jnp.sum(q * q, axis=1, keepdims=True)
    pp = jnp.sum(p * p, axis=0, keepdims=True)
    d = qq + pp - 2.0 * (q @ p)
    iota = lax.broadcasted_iota(jnp.int32, d.shape, 1)
    cols = []
    for _ in range(K):
        m = jnp.min(d, axis=1, keepdims=True)
        sel = jnp.min(jnp.where(d == m, iota, N), axis=1, keepdims=True)
        cols.append(sel)
        d = jnp.where(iota == sel, jnp.float32(np.inf), d)
    o_ref[0] = jnp.concatenate(cols, axis=1)


def _fps_body(B, M, N, xr_ref, xt_ref, o_ref, dist_ref, sel_ref):
    iota = lax.broadcasted_iota(jnp.int32, (B, N), 1)
    dist_ref[...] = jnp.full((B, N), 1e10, jnp.float32)
    sel_ref[...] = jnp.zeros((B, 1), jnp.int32)

    def body(i, carry):
        rows = []
        for b in range(B):
            far = sel_ref[b, 0]
            o_ref[b, pl.ds(i, 1), :] = xr_ref[b, pl.ds(far, 1), :]
            cx = xr_ref[b, far, 0]
            cy = xr_ref[b, far, 1]
            cz = xr_ref[b, far, 2]
            s0 = xt_ref[b, 0:1, :] - cx
            s1 = xt_ref[b, 1:2, :] - cy
            s2 = xt_ref[b, 2:3, :] - cz
            d2 = s0 * s0 + s1 * s1
            d2 = d2 + s2 * s2
            rows.append(d2)
        dist = jnp.minimum(dist_ref[...], jnp.concatenate(rows, axis=0))
        dist_ref[...] = dist
        m = jnp.max(dist, axis=1, keepdims=True)
        sel_ref[...] = jnp.min(jnp.where(dist == m, iota, N), axis=1,
                               keepdims=True).astype(jnp.int32)
        return carry

    lax.fori_loop(0, M, body, jnp.int32(0))


def _attn_body(K, g_ref, q_ref, xz_ref, pre_ref, fdw1, fdb1, fdw2, fdb2,
               fgw1, fgb1, fgw2, fgb2, fc2w, fc2b, o_ref):
    q = q_ref[0]
    xz = xz_ref[0]
    attn = []
    vp = []
    for j in range(K):
        kj = g_ref[0, j, :, 0:256]
        vj = g_ref[0, j, :, 256:512]
        nx = g_ref[0, j, :, 512:528]
        rel = xz - nx
        h = jnp.maximum(rel @ fdw1[...] + fdb1[...], 0.0)
        pos = h @ fdw2[...] + fdb2[...]
        gj = jnp.maximum((q - kj + pos) @ fgw1[...] + fgb1[...], 0.0)
        aj = (gj @ fgw2[...] + fgb2[...]) * jnp.float32(1.0 / 16.0)
        attn.append(aj)
        vp.append(vj + pos)
    m = attn[0]
    for j in range(1, K):
        m = jnp.maximum(m, attn[j])
    num = None
    den = None
    for j in range(K):
        e = jnp.exp(attn[j] - m)
        den = e if den is None else den + e
        c = e * vp[j]
        num = c if num is None else num + c
    res = num / den
    o_ref[0] = res @ fc2w[...] + fc2b[...] + pre_ref[0]


def _td_body(K, d, g_ref, nx_ref, w1a, w1b, b1, g1, be1, w2, b2, g2, be2,
             o_ref):
    sq = np.float32(np.sqrt(1.0 + 1e-05))
    nx = nx_ref[0]
    acc = None
    for j in range(K):
        gp = g_ref[0, j, :, 0:d]
        gx = g_ref[0, j, :, d:d + 16]
        norm = gx - nx
        y = norm @ w1a[...] + gp @ w1b[...] + b1[...]
        y = jnp.maximum((y / sq) * g1[...] + be1[...], 0.0)
        y = y @ w2[...] + b2[...]
        y = jnp.maximum((y / sq) * g2[...] + be2[...], 0.0)
        acc = y if acc is None else jnp.maximum(acc, y)
    o_ref[0] = acc


def _full(shape):
    nd = len(shape)
    return pl.BlockSpec(shape, lambda *_: (0,) * nd)


def _input_mlp(xf, w1, b1, w2, b2):
    R = xf.shape[0]
    t = min(R, 1024)
    return pl.pallas_call(
        _mlp_in_body,
        grid=(R // t,),
        in_specs=[
            pl.BlockSpec((t, 3), lambda i: (i, 0)),
            _full(w1.shape), _full(b1.shape), _full(w2.shape), _full(b2.shape),
        ],
        out_specs=pl.BlockSpec((t, 32), lambda i: (i, 0)),
        out_shape=jax.ShapeDtypeStruct((R, 32), jnp.float32),
        interpret=_INTERPRET,
    )(xf, w1, b1, w2, b2)


def _prep(featsf, xz16f, p):
    R, d = featsf.shape
    t = min(R, 512)
    fc1b = p['fc1_b'].reshape(1, -1)
    q, tab = pl.pallas_call(
        _prep_body,
        grid=(R // t,),
        in_specs=[
            pl.BlockSpec((t, d), lambda i: (i, 0)),
            pl.BlockSpec((t, 16), lambda i: (i, 0)),
            _full((d, 256)), _full((1, 256)),
            _full((256, 256)), _full((256, 256)), _full((256, 256)),
        ],
        out_specs=[
            pl.BlockSpec((t, 256), lambda i: (i, 0)),
            pl.BlockSpec((t, 640), lambda i: (i, 0)),
        ],
        out_shape=[
            jax.ShapeDtypeStruct((R, 256), jnp.float32),
            jax.ShapeDtypeStruct((R, 640), jnp.float32),
        ],
        interpret=_INTERPRET,
    )(featsf, xz16f, p['fc1_w'], fc1b, p['wq'], p['wk'], p['wv'])
    return q, tab


def _knn(q16, pT16, K):
    B, M, _ = q16.shape
    N = pT16.shape[2]
    t = min(M, 256)
    return pl.pallas_call(
        functools.partial(_knn_body, K, N),
        grid=(B, M // t),
        in_specs=[
            pl.BlockSpec((1, t, 16), lambda b, i: (b, i, 0)),
            pl.BlockSpec((1, 16, N), lambda b, i: (b, 0, 0)),
        ],
        out_specs=pl.BlockSpec((1, t, K), lambda b, i: (b, i, 0)),
        out_shape=jax.ShapeDtypeStruct((B, M, K), jnp.int32),
        interpret=_INTERPRET,
    )(q16, pT16)


def _fps(xz16, xzT16, M):
    B, N, _ = xz16.shape
    return pl.pallas_call(
        functools.partial(_fps_body, B, M, N),
        grid=(1,),
        in_specs=[
            pl.BlockSpec((B, N, 16), lambda i: (0, 0, 0)),
            pl.BlockSpec((B, 16, N), lambda i: (0, 0, 0)),
        ],
        out_specs=pl.BlockSpec((B, M, 16), lambda i: (0, 0, 0)),
        out_shape=jax.ShapeDtypeStruct((B, M, 16), jnp.float32),
        scratch_shapes=[pltpu.VMEM((B, N), jnp.float32),
                        pltpu.VMEM((B, 1), jnp.int32)],
        interpret=_INTERPRET,
    )(xz16, xzT16)


def _attn(gath, q, xz16, pre, p, K):
    B, M, d = pre.shape
    t = min(M, 128)
    fdw1 = jnp.concatenate(
        [p['fd_w1'], jnp.zeros((13, 256), jnp.float32)], axis=0)
    args = [gath, q, xz16, pre, fdw1,
            p['fd_b1'].reshape(1, -1), p['fd_w2'], p['fd_b2'].reshape(1, -1),
            p['fg_w1'], p['fg_b1'].reshape(1, -1),
            p['fg_w2'], p['fg_b2'].reshape(1, -1),
            p['fc2_w'], p['fc2_b'].reshape(1, -1)]
    return pl.pallas_call(
        functools.partial(_attn_body, K),
        grid=(B, M // t),
        in_specs=[
            pl.BlockSpec((1, K, t, 640), lambda b, i: (b, 0, i, 0)),
            pl.BlockSpec((1, t, 256), lambda b, i: (b, i, 0)),
            pl.BlockSpec((1, t, 16), lambda b, i: (b, i, 0)),
            pl.BlockSpec((1, t, d), lambda b, i: (b, i, 0)),
            _full((16, 256)), _full((1, 256)),
            _full((256, 256)), _full((1, 256)),
            _full((256, 256)), _full((1, 256)),
            _full((256, 256)), _full((1, 256)),
            _full((256, d)), _full((1, d)),
        ],
        out_specs=pl.BlockSpec((1, t, d), lambda b, i: (b, i, 0)),
        out_shape=jax.ShapeDtypeStruct((B, M, d), jnp.float32),
        interpret=_INTERPRET,
    )(*args)


def _td_conv(gath, nx16, p, d):
    B, K, M, D = gath.shape
    ch = p['convs'][0][0].shape[1]
    t = min(M, 128)
    (w1, b1, g1, be1), (w2, b2, g2, be2) = p['convs']
    w1a = jnp.concatenate(
        [w1[:3], jnp.zeros((13, ch), jnp.float32)], axis=0)
    w1b = w1[3:]
    args = [gath, nx16, w1a, w1b, b1.reshape(1, -1), g1.reshape(1, -1),
            be1.reshape(1, -1), w2, b2.reshape(1, -1), g2.reshape(1, -1),
            be2.reshape(1, -1)]
    return pl.pallas_call(
        functools.partial(_td_body, K, d),
        grid=(B, M // t),
        in_specs=[
            pl.BlockSpec((1, K, t, D), lambda b, i: (b, 0, i, 0)),
            pl.BlockSpec((1, t, 16), lambda b, i: (b, i, 0)),
            _full((16, ch)), _full((d, ch)), _full((1, ch)), _full((1, ch)),
            _full((1, ch)), _full((ch, ch)), _full((1, ch)), _full((1, ch)),
            _full((1, ch)),
        ],
        out_specs=pl.BlockSpec((1, t, ch), lambda b, i: (b, i, 0)),
        out_shape=jax.ShapeDtypeStruct((B, M, ch), jnp.float32),
        interpret=_INTERPRET,
    )(*args)


def _flat_idx(idx, N):
    B = idx.shape[0]
    off = (jnp.arange(B, dtype=jnp.int32) * N)[:, None, None]
    return (jnp.transpose(idx, (0, 2, 1)) + off).reshape(-1)


def _transformer(xz16, xzT16, feats, p, K):
    B, M, d = feats.shape
    idx = _knn(xz16, xzT16, K)
    q, tab = _prep(feats.reshape(B * M, d), xz16.reshape(B * M, 16), p)
    gath = _gather(tab, _flat_idx(idx, M)).reshape(B, K, M, 640)
    return _attn(gath, q.reshape(B, M, 256), xz16, feats, p, K)


def _transition(xz16, xzT16, points, M, p):
    B, N, d = points.shape
    new16 = _fps(xz16, xzT16, M)
    idx = _knn(new16, xzT16, 16)
    dpad = -(-(d + 16) // 128) * 128
    tab = jnp.concatenate(
        [points.reshape(B * N, d), xz16.reshape(B * N, 16),
         jnp.zeros((B * N, dpad - d - 16), jnp.float32)], axis=1)
    gath = _gather(tab, _flat_idx(idx, N)).reshape(B, 16, M, dpad)
    y = _td_conv(gath, new16, p, d)
    return new16, y


def kernel(x, params):
    B, N0, _ = x.shape
    xz16 = jnp.pad(x[..., :3], ((0, 0), (0, 0), (0, 13)))
    xzT16 = jnp.swapaxes(xz16, 1, 2)
    h = _input_mlp(x.reshape(B * N0, 3), params['in_w1'],
                   params['in_b1'].reshape(1, -1), params['in_w2'],
                   params['in_b2'].reshape(1, -1)).reshape(B, N0, 32)
    pts = _transformer(xz16, xzT16, h, params['t0'], 16)
    for i in range(4):
        M = N0 // 4 ** (i + 1)
        xz16, pts = _transition(xz16, xzT16, pts, M, params['td' + str(i)])
        xzT16 = jnp.swapaxes(xz16, 1, 2)
        pts = _transformer(xz16, xzT16, pts, params['t' + str(i + 1)],
                           min(16, M))
    return pts


# speedup vs baseline: 1.3101x; 1.0027x over previous
import functools

import jax
import jax.numpy as jnp
import numpy as np
from jax import lax
from jax.experimental import pallas as pl
from jax.experimental.pallas import tpu as pltpu
from jax.experimental.pallas import tpu_sc as plsc

_INTERPRET = False


def _sc_gather(table, idx):
    G = idx.shape[0]
    D = table.shape[1]
    info = plsc.get_sparse_core_info()
    nc, ns = info.num_cores, info.num_subcores
    nw = nc * ns
    assert G % nw == 0, (G, nw)
    per_w = G // nw
    chunk = 8
    while chunk * 2 <= per_w and chunk * 2 <= 128 and (chunk * 4) * (D + 1) <= 110000:
        chunk *= 2
    chunk = min(chunk, per_w)
    nch = per_w // chunk
    mesh = plsc.VectorSubcoreMesh(core_axis_name="c", subcore_axis_name="s")

    @functools.partial(
        pl.kernel,
        mesh=mesh,
        out_type=jax.ShapeDtypeStruct((G, D), jnp.float32),
        scratch_types=[
            pltpu.VMEM((chunk,), jnp.int32),
            pltpu.VMEM((chunk,), jnp.int32),
            pltpu.VMEM((chunk, D), jnp.float32),
            pltpu.VMEM((chunk, D), jnp.float32),
            pltpu.SemaphoreType.DMA,
            pltpu.SemaphoreType.DMA,
            pltpu.SemaphoreType.DMA,
            pltpu.SemaphoreType.DMA,
            pltpu.SemaphoreType.DMA,
        ],
    )
    def gk(tab_hbm, idx_hbm, out_hbm, idx0, idx1, rows0, rows1,
           isem0, isem1, gsem, osem0, osem1):
        idx_v = [idx0, idx1]
        rows_v = [rows0, rows1]
        isem = [isem0, isem1]
        osem = [osem0, osem1]
        wid = lax.axis_index("s") * nc + lax.axis_index("c")
        base = wid * per_w
        h_i = [None, None]
        h_o = [None, None]
        h_i[0] = pltpu.async_copy(
            idx_hbm.at[pl.ds(base, chunk)], idx_v[0], isem[0])
        for ci in range(nch):
            cur = ci % 2
            if ci + 1 < nch:
                off_n = base + (ci + 1) * chunk
                h_i[1 - cur] = pltpu.async_copy(
                    idx_hbm.at[pl.ds(off_n, chunk)], idx_v[1 - cur],
                    isem[1 - cur])
            h_i[cur].wait()
            if h_o[cur] is not None:
                h_o[cur].wait()
            pltpu.async_copy(tab_hbm.at[idx_v[cur]], rows_v[cur], gsem).wait()
            off = base + ci * chunk
            h_o[cur] = pltpu.async_copy(
                rows_v[cur], out_hbm.at[pl.ds(off, chunk)], osem[cur])
        for h in h_o:
            if h is not None:
                h.wait()

    return gk(table, idx)


_gather = _sc_gather


def _mlp_in_body(x_ref, w1, b1, w2, b2, o_ref):
    h = jnp.maximum(x_ref[...] @ w1[...] + b1[...], 0.0)
    o_ref[...] = h @ w2[...] + b2[...]


def _prep_body(f_ref, xz_ref, fc1w, fc1b, wq, wk, wv, q_ref, tab_ref):
    xx = f_ref[...] @ fc1w[...] + fc1b[...]
    q_ref[...] = xx @ wq[...]
    tab_ref[:, 0:256] = xx @ wk[...]
    tab_ref[:, 256:512] = xx @ wv[...]
    tab_ref[:, 512:528] = xz_ref[...]
    tab_ref[:, 528:640] = jnp.zeros_like(tab_ref[:, 528:640])


def _knn_body(K, N, q_ref, p_ref, o_ref):
    q = q_ref[0]
    p = p_ref[0]
    qq = jnp.sum(q * q, axis=1, keepdims=True)
    pp = jnp.sum(p * p, axis=0, keepdims=True)
    d = qq + pp - 2.0 * (q @ p)
    iota = lax.broadcasted_iota(jnp.int32, d.shape, 1)
    cols = []
    for _ in range(K):
        m = jnp.min(d, axis=1, keepdims=True)
        sel = jnp.min(jnp.where(d == m, iota, N), axis=1, keepdims=True)
        cols.append(sel)
        d = jnp.where(iota == sel, jnp.float32(np.inf), d)
    o_ref[0] = jnp.concatenate(cols, axis=1)


def _fps_body(B, M, N, xr_ref, xt_ref, o_ref, dist_ref, sel_ref):
    iota = lax.broadcasted_iota(jnp.int32, (B, N), 1)
    dist_ref[...] = jnp.full((B, N), 1e10, jnp.float32)
    sel_ref[...] = jnp.zeros((B, 1), jnp.int32)

    def body(i, carry):
        rows = []
        for b in range(B):
            far = sel_ref[b, 0]
            o_ref[b, pl.ds(i, 1), :] = xr_ref[b, pl.ds(far, 1), :]
            cx = xr_ref[b, far, 0]
            cy = xr_ref[b, far, 1]
            cz = xr_ref[b, far, 2]
            s0 = xt_ref[b, 0:1, :] - cx
            s1 = xt_ref[b, 1:2, :] - cy
            s2 = xt_ref[b, 2:3, :] - cz
            d2 = s0 * s0 + s1 * s1
            d2 = d2 + s2 * s2
            rows.append(d2)
        dist = jnp.minimum(dist_ref[...], jnp.concatenate(rows, axis=0))
        dist_ref[...] = dist
        m = jnp.max(dist, axis=1, keepdims=True)
        sel_ref[...] = jnp.min(jnp.where(dist == m, iota, N), axis=1,
                               keepdims=True).astype(jnp.int32)
        return carry

    lax.fori_loop(0, M, body, jnp.int32(0))


def _attn_body(K, g_ref, q_ref, xz_ref, pre_ref, fdw1, fdb1, fdw2, fdb2,
               fgw1, fgb1, fgw2, fgb2, fc2w, fc2b, o_ref):
    q = q_ref[0]
    xz = xz_ref[0]
    attn = []
    vp = []
    for j in range(K):
        kj = g_ref[0, j, :, 0:256]
        vj = g_ref[0, j, :, 256:512]
        nx = g_ref[0, j, :, 512:528]
        rel = xz - nx
        h = jnp.maximum(rel @ fdw1[...] + fdb1[...], 0.0)
        pos = h @ fdw2[...] + fdb2[...]
        gj = jnp.maximum((q - kj + pos) @ fgw1[...] + fgb1[...], 0.0)
        aj = (gj @ fgw2[...] + fgb2[...]) * jnp.float32(1.0 / 16.0)
        attn.append(aj)
        vp.append(vj + pos)
    m = attn[0]
    for j in range(1, K):
        m = jnp.maximum(m, attn[j])
    num = None
    den = None
    for j in range(K):
        e = jnp.exp(attn[j] - m)
        den = e if den is None else den + e
        c = e * vp[j]
        num = c if num is None else num + c
    res = num / den
    o_ref[0] = res @ fc2w[...] + fc2b[...] + pre_ref[0]


def _td_body(K, d, g_ref, nx_ref, w1a, w1b, b1, g1, be1, w2, b2, g2, be2,
             o_ref):
    sq = np.float32(np.sqrt(1.0 + 1e-05))
    nx = nx_ref[0]
    acc = None
    for j in range(K):
        gp = g_ref[0, j, :, 0:d]
        gx = g_ref[0, j, :, d:d + 16]
        norm = gx - nx
        y = norm @ w1a[...] + gp @ w1b[...] + b1[...]
        y = jnp.maximum((y / sq) * g1[...] + be1[...], 0.0)
        y = y @ w2[...] + b2[...]
        y = jnp.maximum((y / sq) * g2[...] + be2[...], 0.0)
        acc = y if acc is None else jnp.maximum(acc, y)
    o_ref[0] = acc


def _full(shape):
    nd = len(shape)
    return pl.BlockSpec(shape, lambda *_: (0,) * nd)


def _input_mlp(xf, w1, b1, w2, b2):
    R = xf.shape[0]
    t = min(R, 1024)
    return pl.pallas_call(
        _mlp_in_body,
        grid=(R // t,),
        in_specs=[
            pl.BlockSpec((t, 3), lambda i: (i, 0)),
            _full(w1.shape), _full(b1.shape), _full(w2.shape), _full(b2.shape),
        ],
        out_specs=pl.BlockSpec((t, 32), lambda i: (i, 0)),
        out_shape=jax.ShapeDtypeStruct((R, 32), jnp.float32),
        interpret=_INTERPRET,
    )(xf, w1, b1, w2, b2)


def _prep(featsf, xz16f, p):
    R, d = featsf.shape
    t = min(R, 512)
    fc1b = p['fc1_b'].reshape(1, -1)
    q, tab = pl.pallas_call(
        _prep_body,
        grid=(R // t,),
        in_specs=[
            pl.BlockSpec((t, d), lambda i: (i, 0)),
            pl.BlockSpec((t, 16), lambda i: (i, 0)),
            _full((d, 256)), _full((1, 256)),
            _full((256, 256)), _full((256, 256)), _full((256, 256)),
        ],
        out_specs=[
            pl.BlockSpec((t, 256), lambda i: (i, 0)),
            pl.BlockSpec((t, 640), lambda i: (i, 0)),
        ],
        out_shape=[
            jax.ShapeDtypeStruct((R, 256), jnp.float32),
            jax.ShapeDtypeStruct((R, 640), jnp.float32),
        ],
        interpret=_INTERPRET,
    )(featsf, xz16f, p['fc1_w'], fc1b, p['wq'], p['wk'], p['wv'])
    return q, tab


def _knn(q16, pT16, K):
    B, M, _ = q16.shape
    N = pT16.shape[2]
    t = min(M, 256)
    return pl.pallas_call(
        functools.partial(_knn_body, K, N),
        grid=(B, M // t),
        in_specs=[
            pl.BlockSpec((1, t, 16), lambda b, i: (b, i, 0)),
            pl.BlockSpec((1, 16, N), lambda b, i: (b, 0, 0)),
        ],
        out_specs=pl.BlockSpec((1, t, K), lambda b, i: (b, i, 0)),
        out_shape=jax.ShapeDtypeStruct((B, M, K), jnp.int32),
        interpret=_INTERPRET,
    )(q16, pT16)


def _fps(xz16, xzT16, M):
    B, N, _ = xz16.shape
    return pl.pallas_call(
        functools.partial(_fps_body, B, M, N),
        grid=(1,),
        in_specs=[
            pl.BlockSpec((B, N, 16), lambda i: (0, 0, 0)),
            pl.BlockSpec((B, 16, N), lambda i: (0, 0, 0)),
        ],
        out_specs=pl.BlockSpec((B, M, 16), lambda i: (0, 0, 0)),
        out_shape=jax.ShapeDtypeStruct((B, M, 16), jnp.float32),
        scratch_shapes=[pltpu.VMEM((B, N), jnp.float32),
                        pltpu.VMEM((B, 1), jnp.int32)],
        interpret=_INTERPRET,
    )(xz16, xzT16)


def _attn(gath, q, xz16, pre, p, K):
    B, M, d = pre.shape
    t = min(M, 128)
    fdw1 = jnp.concatenate(
        [p['fd_w1'], jnp.zeros((13, 256), jnp.float32)], axis=0)
    args = [gath, q, xz16, pre, fdw1,
            p['fd_b1'].reshape(1, -1), p['fd_w2'], p['fd_b2'].reshape(1, -1),
            p['fg_w1'], p['fg_b1'].reshape(1, -1),
            p['fg_w2'], p['fg_b2'].reshape(1, -1),
            p['fc2_w'], p['fc2_b'].reshape(1, -1)]
    return pl.pallas_call(
        functools.partial(_attn_body, K),
        grid=(B, M // t),
        in_specs=[
            pl.BlockSpec((1, K, t, 640), lambda b, i: (b, 0, i, 0)),
            pl.BlockSpec((1, t, 256), lambda b, i: (b, i, 0)),
            pl.BlockSpec((1, t, 16), lambda b, i: (b, i, 0)),
            pl.BlockSpec((1, t, d), lambda b, i: (b, i, 0)),
            _full((16, 256)), _full((1, 256)),
            _full((256, 256)), _full((1, 256)),
            _full((256, 256)), _full((1, 256)),
            _full((256, 256)), _full((1, 256)),
            _full((256, d)), _full((1, d)),
        ],
        out_specs=pl.BlockSpec((1, t, d), lambda b, i: (b, i, 0)),
        out_shape=jax.ShapeDtypeStruct((B, M, d), jnp.float32),
        interpret=_INTERPRET,
    )(*args)


def _td_conv(gath, nx16, p, d):
    B, K, M, D = gath.shape
    ch = p['convs'][0][0].shape[1]
    t = min(M, 128)
    (w1, b1, g1, be1), (w2, b2, g2, be2) = p['convs']
    w1a = jnp.concatenate(
        [w1[:3], jnp.zeros((13, ch), jnp.float32)], axis=0)
    w1b = w1[3:]
    args = [gath, nx16, w1a, w1b, b1.reshape(1, -1), g1.reshape(1, -1),
            be1.reshape(1, -1), w2, b2.reshape(1, -1), g2.reshape(1, -1),
            be2.reshape(1, -1)]
    return pl.pallas_call(
        functools.partial(_td_body, K, d),
        grid=(B, M // t),
        in_specs=[
            pl.BlockSpec((1, K, t, D), lambda b, i: (b, 0, i, 0)),
            pl.BlockSpec((1, t, 16), lambda b, i: (b, i, 0)),
            _full((16, ch)), _full((d, ch)), _full((1, ch)), _full((1, ch)),
            _full((1, ch)), _full((ch, ch)), _full((1, ch)), _full((1, ch)),
            _full((1, ch)),
        ],
        out_specs=pl.BlockSpec((1, t, ch), lambda b, i: (b, i, 0)),
        out_shape=jax.ShapeDtypeStruct((B, M, ch), jnp.float32),
        interpret=_INTERPRET,
    )(*args)


def _flat_idx(idx, N):
    B = idx.shape[0]
    off = (jnp.arange(B, dtype=jnp.int32) * N)[:, None, None]
    return (jnp.transpose(idx, (0, 2, 1)) + off).reshape(-1)


def _transformer(xz16, xzT16, feats, p, K):
    B, M, d = feats.shape
    idx = _knn(xz16, xzT16, K)
    q, tab = _prep(feats.reshape(B * M, d), xz16.reshape(B * M, 16), p)
    gath = _gather(tab, _flat_idx(idx, M)).reshape(B, K, M, 640)
    return _attn(gath, q.reshape(B, M, 256), xz16, feats, p, K)


def _transition(xz16, xzT16, points, M, p):
    B, N, d = points.shape
    new16 = _fps(xz16, xzT16, M)
    idx = _knn(new16, xzT16, 16)
    dpad = -(-(d + 16) // 128) * 128
    tab = jnp.concatenate(
        [points.reshape(B * N, d), xz16.reshape(B * N, 16),
         jnp.zeros((B * N, dpad - d - 16), jnp.float32)], axis=1)
    gath = _gather(tab, _flat_idx(idx, N)).reshape(B, 16, M, dpad)
    y = _td_conv(gath, new16, p, d)
    return new16, y


def kernel(x, params):
    B, N0, _ = x.shape
    xz16 = jnp.pad(x[..., :3], ((0, 0), (0, 0), (0, 13)))
    xzT16 = jnp.swapaxes(xz16, 1, 2)
    h = _input_mlp(x.reshape(B * N0, 3), params['in_w1'],
                   params['in_b1'].reshape(1, -1), params['in_w2'],
                   params['in_b2'].reshape(1, -1)).reshape(B, N0, 32)
    pts = _transformer(xz16, xzT16, h, params['t0'], 16)
    for i in range(4):
        M = N0 // 4 ** (i + 1)
        xz16, pts = _transition(xz16, xzT16, pts, M, params['td' + str(i)])
        xzT16 = jnp.swapaxes(xz16, 1, 2)
        pts = _transformer(xz16, xzT16, pts, params['t' + str(i + 1)],
                           min(16, M))
    return pts


# speedup vs baseline: 1.3167x; 1.0050x over previous
import functools

import jax
import jax.numpy as jnp
import numpy as np
from jax import lax
from jax.experimental import pallas as pl
from jax.experimental.pallas import tpu as pltpu
from jax.experimental.pallas import tpu_sc as plsc

_INTERPRET = False


def _sc_gather(table, idx):
    G = idx.shape[0]
    D = table.shape[1]
    info = plsc.get_sparse_core_info()
    nc, ns = info.num_cores, info.num_subcores
    nw = nc * ns
    assert G % nw == 0, (G, nw)
    per_w = G // nw
    chunk = 8
    while chunk * 2 <= per_w and chunk * 2 <= 128 and (chunk * 4) * (D + 1) <= 110000:
        chunk *= 2
    chunk = min(chunk, per_w)
    nch = per_w // chunk
    mesh = plsc.VectorSubcoreMesh(core_axis_name="c", subcore_axis_name="s")

    @functools.partial(
        pl.kernel,
        mesh=mesh,
        out_type=jax.ShapeDtypeStruct((G, D), jnp.float32),
        scratch_types=[
            pltpu.VMEM((chunk,), jnp.int32),
            pltpu.VMEM((chunk,), jnp.int32),
            pltpu.VMEM((chunk, D), jnp.float32),
            pltpu.VMEM((chunk, D), jnp.float32),
            pltpu.SemaphoreType.DMA,
            pltpu.SemaphoreType.DMA,
            pltpu.SemaphoreType.DMA,
            pltpu.SemaphoreType.DMA,
            pltpu.SemaphoreType.DMA,
        ],
    )
    def gk(tab_hbm, idx_hbm, out_hbm, idx0, idx1, rows0, rows1,
           isem0, isem1, gsem, osem0, osem1):
        idx_v = [idx0, idx1]
        rows_v = [rows0, rows1]
        isem = [isem0, isem1]
        osem = [osem0, osem1]
        wid = lax.axis_index("s") * nc + lax.axis_index("c")
        base = wid * per_w
        h_i = [None, None]
        h_o = [None, None]
        h_i[0] = pltpu.async_copy(
            idx_hbm.at[pl.ds(base, chunk)], idx_v[0], isem[0])
        for ci in range(nch):
            cur = ci % 2
            if ci + 1 < nch:
                off_n = base + (ci + 1) * chunk
                h_i[1 - cur] = pltpu.async_copy(
                    idx_hbm.at[pl.ds(off_n, chunk)], idx_v[1 - cur],
                    isem[1 - cur])
            h_i[cur].wait()
            if h_o[cur] is not None:
                h_o[cur].wait()
            pltpu.async_copy(tab_hbm.at[idx_v[cur]], rows_v[cur], gsem).wait()
            off = base + ci * chunk
            h_o[cur] = pltpu.async_copy(
                rows_v[cur], out_hbm.at[pl.ds(off, chunk)], osem[cur])
        for h in h_o:
            if h is not None:
                h.wait()

    return gk(table, idx)


_gather = _sc_gather


def _mlp_in_body(x_ref, w1, b1, w2, b2, o_ref):
    h = jnp.maximum(x_ref[...] @ w1[...] + b1[...], 0.0)
    o_ref[...] = h @ w2[...] + b2[...]


def _prep_body(f_ref, xz_ref, fc1w, fc1b, wq, wk, wv, q_ref, tab_ref):
    xx = f_ref[...] @ fc1w[...] + fc1b[...]
    q_ref[...] = xx @ wq[...]
    tab_ref[:, 0:256] = xx @ wk[...]
    tab_ref[:, 256:512] = xx @ wv[...]
    tab_ref[:, 512:528] = xz_ref[...]
    tab_ref[:, 528:640] = jnp.zeros_like(tab_ref[:, 528:640])


def _knn_body(K, N, q_ref, p_ref, o_ref):
    q = q_ref[0]
    p = p_ref[0]
    qq = jnp.sum(q * q, axis=1, keepdims=True)
    pp = jnp.sum(p * p, axis=0, keepdims=True)
    d = qq + pp - 2.0 * (q @ p)
    iota = lax.broadcasted_iota(jnp.int32, d.shape, 1)
    cols = []
    for _ in range(K):
        m = jnp.min(d, axis=1, keepdims=True)
        sel = jnp.min(jnp.where(d == m, iota, N), axis=1, keepdims=True)
        cols.append(sel)
        d = jnp.where(iota == sel, jnp.float32(np.inf), d)
    o_ref[0] = jnp.concatenate(cols, axis=1)


def _fps_body(B, M, N, xr_ref, xt_ref, o_ref, dist_ref, sel_ref):
    iota = lax.broadcasted_iota(jnp.int32, (B, N), 1)
    dist_ref[...] = jnp.full((B, N), 1e10, jnp.float32)
    sel_ref[...] = jnp.zeros((B, 1), jnp.int32)

    def body(i, carry):
        rows = []
        for b in range(B):
            far = sel_ref[b, 0]
            o_ref[b, pl.ds(i, 1), :] = xr_ref[b, pl.ds(far, 1), :]
            cx = xr_ref[b, far, 0]
            cy = xr_ref[b, far, 1]
            cz = xr_ref[b, far, 2]
            s0 = xt_ref[b, 0:1, :] - cx
            s1 = xt_ref[b, 1:2, :] - cy
            s2 = xt_ref[b, 2:3, :] - cz
            d2 = s0 * s0 + s1 * s1
            d2 = d2 + s2 * s2
            rows.append(d2)
        dist = jnp.minimum(dist_ref[...], jnp.concatenate(rows, axis=0))
        dist_ref[...] = dist
        m = jnp.max(dist, axis=1, keepdims=True)
        sel_ref[...] = jnp.min(jnp.where(dist == m, iota, N), axis=1,
                               keepdims=True).astype(jnp.int32)
        return carry

    lax.fori_loop(0, M, body, jnp.int32(0))


def _attn_body(K, g_ref, q_ref, xz_ref, pre_ref, fdw1, fdb1, fdw2, fdb2,
               fgw1, fgb1, fgw2, fgb2, fc2w, fc2b, o_ref):
    q = q_ref[0]
    xz = xz_ref[0]
    m = None
    den = None
    num = None
    for j in range(K):
        kj = g_ref[0, j, :, 0:256]
        vj = g_ref[0, j, :, 256:512]
        nx = g_ref[0, j, :, 512:528]
        rel = xz - nx
        h = jnp.maximum(rel @ fdw1[...] + fdb1[...], 0.0)
        pos = h @ fdw2[...] + fdb2[...]
        gj = jnp.maximum((q - kj + pos) @ fgw1[...] + fgb1[...], 0.0)
        aj = (gj @ fgw2[...] + fgb2[...]) * jnp.float32(1.0 / 16.0)
        vpj = vj + pos
        if j == 0:
            m = aj
            den = jnp.ones_like(aj)
            num = vpj
        else:
            m_new = jnp.maximum(m, aj)
            scale = jnp.exp(m - m_new)
            e = jnp.exp(aj - m_new)
            den = den * scale + e
            num = num * scale + e * vpj
            m = m_new
    res = num / den
    o_ref[0] = res @ fc2w[...] + fc2b[...] + pre_ref[0]


def _td_body(K, d, g_ref, nx_ref, w1a, w1b, b1, g1, be1, w2, b2, g2, be2,
             o_ref):
    sq = np.float32(np.sqrt(1.0 + 1e-05))
    nx = nx_ref[0]
    acc = None
    for j in range(K):
        gp = g_ref[0, j, :, 0:d]
        gx = g_ref[0, j, :, d:d + 16]
        norm = gx - nx
        y = norm @ w1a[...] + gp @ w1b[...] + b1[...]
        y = jnp.maximum((y / sq) * g1[...] + be1[...], 0.0)
        y = y @ w2[...] + b2[...]
        y = jnp.maximum((y / sq) * g2[...] + be2[...], 0.0)
        acc = y if acc is None else jnp.maximum(acc, y)
    o_ref[0] = acc


def _full(shape):
    nd = len(shape)
    return pl.BlockSpec(shape, lambda *_: (0,) * nd)


def _input_mlp(xf, w1, b1, w2, b2):
    R = xf.shape[0]
    t = min(R, 1024)
    return pl.pallas_call(
        _mlp_in_body,
        grid=(R // t,),
        in_specs=[
            pl.BlockSpec((t, 3), lambda i: (i, 0)),
            _full(w1.shape), _full(b1.shape), _full(w2.shape), _full(b2.shape),
        ],
        out_specs=pl.BlockSpec((t, 32), lambda i: (i, 0)),
        out_shape=jax.ShapeDtypeStruct((R, 32), jnp.float32),
        interpret=_INTERPRET,
    )(xf, w1, b1, w2, b2)


def _prep(featsf, xz16f, p):
    R, d = featsf.shape
    t = min(R, 512)
    fc1b = p['fc1_b'].reshape(1, -1)
    q, tab = pl.pallas_call(
        _prep_body,
        grid=(R // t,),
        in_specs=[
            pl.BlockSpec((t, d), lambda i: (i, 0)),
            pl.BlockSpec((t, 16), lambda i: (i, 0)),
            _full((d, 256)), _full((1, 256)),
            _full((256, 256)), _full((256, 256)), _full((256, 256)),
        ],
        out_specs=[
            pl.BlockSpec((t, 256), lambda i: (i, 0)),
            pl.BlockSpec((t, 640), lambda i: (i, 0)),
        ],
        out_shape=[
            jax.ShapeDtypeStruct((R, 256), jnp.float32),
            jax.ShapeDtypeStruct((R, 640), jnp.float32),
        ],
        interpret=_INTERPRET,
    )(featsf, xz16f, p['fc1_w'], fc1b, p['wq'], p['wk'], p['wv'])
    return q, tab


def _knn(q16, pT16, K):
    B, M, _ = q16.shape
    N = pT16.shape[2]
    t = min(M, 256)
    return pl.pallas_call(
        functools.partial(_knn_body, K, N),
        grid=(B, M // t),
        in_specs=[
            pl.BlockSpec((1, t, 16), lambda b, i: (b, i, 0)),
            pl.BlockSpec((1, 16, N), lambda b, i: (b, 0, 0)),
        ],
        out_specs=pl.BlockSpec((1, t, K), lambda b, i: (b, i, 0)),
        out_shape=jax.ShapeDtypeStruct((B, M, K), jnp.int32),
        interpret=_INTERPRET,
    )(q16, pT16)


def _fps(xz16, xzT16, M):
    B, N, _ = xz16.shape
    return pl.pallas_call(
        functools.partial(_fps_body, B, M, N),
        grid=(1,),
        in_specs=[
            pl.BlockSpec((B, N, 16), lambda i: (0, 0, 0)),
            pl.BlockSpec((B, 16, N), lambda i: (0, 0, 0)),
        ],
        out_specs=pl.BlockSpec((B, M, 16), lambda i: (0, 0, 0)),
        out_shape=jax.ShapeDtypeStruct((B, M, 16), jnp.float32),
        scratch_shapes=[pltpu.VMEM((B, N), jnp.float32),
                        pltpu.VMEM((B, 1), jnp.int32)],
        interpret=_INTERPRET,
    )(xz16, xzT16)


def _attn(gath, q, xz16, pre, p, K):
    B, M, d = pre.shape
    t = min(M, 128)
    fdw1 = jnp.concatenate(
        [p['fd_w1'], jnp.zeros((13, 256), jnp.float32)], axis=0)
    args = [gath, q, xz16, pre, fdw1,
            p['fd_b1'].reshape(1, -1), p['fd_w2'], p['fd_b2'].reshape(1, -1),
            p['fg_w1'], p['fg_b1'].reshape(1, -1),
            p['fg_w2'], p['fg_b2'].reshape(1, -1),
            p['fc2_w'], p['fc2_b'].reshape(1, -1)]
    return pl.pallas_call(
        functools.partial(_attn_body, K),
        grid=(B, M // t),
        in_specs=[
            pl.BlockSpec((1, K, t, 640), lambda b, i: (b, 0, i, 0)),
            pl.BlockSpec((1, t, 256), lambda b, i: (b, i, 0)),
            pl.BlockSpec((1, t, 16), lambda b, i: (b, i, 0)),
            pl.BlockSpec((1, t, d), lambda b, i: (b, i, 0)),
            _full((16, 256)), _full((1, 256)),
            _full((256, 256)), _full((1, 256)),
            _full((256, 256)), _full((1, 256)),
            _full((256, 256)), _full((1, 256)),
            _full((256, d)), _full((1, d)),
        ],
        out_specs=pl.BlockSpec((1, t, d), lambda b, i: (b, i, 0)),
        out_shape=jax.ShapeDtypeStruct((B, M, d), jnp.float32),
        interpret=_INTERPRET,
    )(*args)


def _td_conv(gath, nx16, p, d):
    B, K, M, D = gath.shape
    ch = p['convs'][0][0].shape[1]
    t = min(M, 128)
    (w1, b1, g1, be1), (w2, b2, g2, be2) = p['convs']
    w1a = jnp.concatenate(
        [w1[:3], jnp.zeros((13, ch), jnp.float32)], axis=0)
    w1b = w1[3:]
    args = [gath, nx16, w1a, w1b, b1.reshape(1, -1), g1.reshape(1, -1),
            be1.reshape(1, -1), w2, b2.reshape(1, -1), g2.reshape(1, -1),
            be2.reshape(1, -1)]
    return pl.pallas_call(
        functools.partial(_td_body, K, d),
        grid=(B, M // t),
        in_specs=[
            pl.BlockSpec((1, K, t, D), lambda b, i: (b, 0, i, 0)),
            pl.BlockSpec((1, t, 16), lambda b, i: (b, i, 0)),
            _full((16, ch)), _full((d, ch)), _full((1, ch)), _full((1, ch)),
            _full((1, ch)), _full((ch, ch)), _full((1, ch)), _full((1, ch)),
            _full((1, ch)),
        ],
        out_specs=pl.BlockSpec((1, t, ch), lambda b, i: (b, i, 0)),
        out_shape=jax.ShapeDtypeStruct((B, M, ch), jnp.float32),
        interpret=_INTERPRET,
    )(*args)


def _flat_idx(idx, N):
    B = idx.shape[0]
    off = (jnp.arange(B, dtype=jnp.int32) * N)[:, None, None]
    return (jnp.transpose(idx, (0, 2, 1)) + off).reshape(-1)


def _transformer(xz16, xzT16, feats, p, K):
    B, M, d = feats.shape
    idx = _knn(xz16, xzT16, K)
    q, tab = _prep(feats.reshape(B * M, d), xz16.reshape(B * M, 16), p)
    gath = _gather(tab, _flat_idx(idx, M)).reshape(B, K, M, 640)
    return _attn(gath, q.reshape(B, M, 256), xz16, feats, p, K)


def _transition(xz16, xzT16, points, M, p):
    B, N, d = points.shape
    new16 = _fps(xz16, xzT16, M)
    idx = _knn(new16, xzT16, 16)
    dpad = -(-(d + 16) // 128) * 128
    tab = jnp.concatenate(
        [points.reshape(B * N, d), xz16.reshape(B * N, 16),
         jnp.zeros((B * N, dpad - d - 16), jnp.float32)], axis=1)
    gath = _gather(tab, _flat_idx(idx, N)).reshape(B, 16, M, dpad)
    y = _td_conv(gath, new16, p, d)
    return new16, y


def kernel(x, params):
    B, N0, _ = x.shape
    xz16 = jnp.pad(x[..., :3], ((0, 0), (0, 0), (0, 13)))
    xzT16 = jnp.swapaxes(xz16, 1, 2)
    h = _input_mlp(x.reshape(B * N0, 3), params['in_w1'],
                   params['in_b1'].reshape(1, -1), params['in_w2'],
                   params['in_b2'].reshape(1, -1)).reshape(B, N0, 32)
    pts = _transformer(xz16, xzT16, h, params['t0'], 16)
    for i in range(4):
        M = N0 // 4 ** (i + 1)
        xz16, pts = _transition(xz16, xzT16, pts, M, params['td' + str(i)])
        xzT16 = jnp.swapaxes(xz16, 1, 2)
        pts = _transformer(xz16, xzT16, pts, params['t' + str(i + 1)],
                           min(16, M))
    return pts


# speedup vs baseline: 1.5833x; 1.2025x over previous
import functools

import jax
import jax.numpy as jnp
import numpy as np
from jax import lax
from jax.experimental import pallas as pl
from jax.experimental.pallas import tpu as pltpu
from jax.experimental.pallas import tpu_sc as plsc

_INTERPRET = False


def _sc_gather(table, idx):
    G = idx.shape[0]
    D = table.shape[1]
    info = plsc.get_sparse_core_info()
    nc, ns = info.num_cores, info.num_subcores
    nw = nc * ns
    assert G % nw == 0, (G, nw)
    per_w = G // nw
    chunk = 8
    while chunk * 2 <= per_w and chunk * 2 <= 128 and (chunk * 4) * (D + 1) <= 110000:
        chunk *= 2
    chunk = min(chunk, per_w)
    nch = per_w // chunk
    mesh = plsc.VectorSubcoreMesh(core_axis_name="c", subcore_axis_name="s")

    @functools.partial(
        pl.kernel,
        mesh=mesh,
        out_type=jax.ShapeDtypeStruct((G, D), jnp.float32),
        scratch_types=[
            pltpu.VMEM((chunk,), jnp.int32),
            pltpu.VMEM((chunk,), jnp.int32),
            pltpu.VMEM((chunk, D), jnp.float32),
            pltpu.VMEM((chunk, D), jnp.float32),
            pltpu.SemaphoreType.DMA,
            pltpu.SemaphoreType.DMA,
            pltpu.SemaphoreType.DMA,
            pltpu.SemaphoreType.DMA,
            pltpu.SemaphoreType.DMA,
        ],
    )
    def gk(tab_hbm, idx_hbm, out_hbm, idx0, idx1, rows0, rows1,
           isem0, isem1, gsem, osem0, osem1):
        idx_v = [idx0, idx1]
        rows_v = [rows0, rows1]
        isem = [isem0, isem1]
        osem = [osem0, osem1]
        wid = lax.axis_index("s") * nc + lax.axis_index("c")
        base = wid * per_w
        h_i = [None, None]
        h_o = [None, None]
        h_i[0] = pltpu.async_copy(
            idx_hbm.at[pl.ds(base, chunk)], idx_v[0], isem[0])
        for ci in range(nch):
            cur = ci % 2
            if ci + 1 < nch:
                off_n = base + (ci + 1) * chunk
                h_i[1 - cur] = pltpu.async_copy(
                    idx_hbm.at[pl.ds(off_n, chunk)], idx_v[1 - cur],
                    isem[1 - cur])
            h_i[cur].wait()
            if h_o[cur] is not None:
                h_o[cur].wait()
            pltpu.async_copy(tab_hbm.at[idx_v[cur]], rows_v[cur], gsem).wait()
            off = base + ci * chunk
            h_o[cur] = pltpu.async_copy(
                rows_v[cur], out_hbm.at[pl.ds(off, chunk)], osem[cur])
        for h in h_o:
            if h is not None:
                h.wait()

    return gk(table, idx)


_gather = _sc_gather


def _mlp_in_body(x_ref, w1, b1, w2, b2, o_ref):
    h = jnp.maximum(x_ref[...] @ w1[...] + b1[...], 0.0)
    o_ref[...] = h @ w2[...] + b2[...]


def _prep_body(f_ref, xz_ref, fc1w, fc1b, wq, wk, wv, q_ref, tab_ref):
    xx = f_ref[...] @ fc1w[...] + fc1b[...]
    q_ref[...] = xx @ wq[...]
    tab_ref[:, 0:256] = xx @ wk[...]
    tab_ref[:, 256:512] = xx @ wv[...]
    tab_ref[:, 512:528] = xz_ref[...]
    tab_ref[:, 528:640] = jnp.zeros_like(tab_ref[:, 528:640])


def _knn_body(K, N, q_ref, p_ref, o_ref):
    q = q_ref[0]
    p = p_ref[0]
    qq = jnp.sum(q * q, axis=1, keepdims=True)
    pp = jnp.sum(p * p, axis=0, keepdims=True)
    d = qq + pp - 2.0 * (q @ p)
    iota = lax.broadcasted_iota(jnp.int32, d.shape, 1)
    cols = []
    for _ in range(K):
        m = jnp.min(d, axis=1, keepdims=True)
        sel = jnp.min(jnp.where(d == m, iota, N), axis=1, keepdims=True)
        cols.append(sel)
        d = jnp.where(iota == sel, jnp.float32(np.inf), d)
    o_ref[0] = jnp.concatenate(cols, axis=1)


def _fps_body(B, M, N, xr_ref, xt_ref, o_ref, dist_ref, sel_ref):
    iota = lax.broadcasted_iota(jnp.int32, (B, N), 1)
    dist_ref[...] = jnp.full((B, N), 1e10, jnp.float32)
    sel_ref[...] = jnp.zeros((B, 1), jnp.int32)

    def body(i, carry):
        rows = []
        for b in range(B):
            far = sel_ref[b, 0]
            o_ref[b, pl.ds(i, 1), :] = xr_ref[b, pl.ds(far, 1), :]
            cx = xr_ref[b, far, 0]
            cy = xr_ref[b, far, 1]
            cz = xr_ref[b, far, 2]
            s0 = xt_ref[b, 0:1, :] - cx
            s1 = xt_ref[b, 1:2, :] - cy
            s2 = xt_ref[b, 2:3, :] - cz
            d2 = s0 * s0 + s1 * s1
            d2 = d2 + s2 * s2
            rows.append(d2)
        dist = jnp.minimum(dist_ref[...], jnp.concatenate(rows, axis=0))
        dist_ref[...] = dist
        m = jnp.max(dist, axis=1, keepdims=True)
        sel_ref[...] = jnp.min(jnp.where(dist == m, iota, N), axis=1,
                               keepdims=True).astype(jnp.int32)
        return carry

    lax.fori_loop(0, M, body, jnp.int32(0))


def _attn_body(K, g_ref, q_ref, xz_ref, pre_ref, fdw1, fdb1, fdw2, fdb2,
               fgw1, fgb1, fgw2, fgb2, fc2w, fc2b, o_ref):
    t = q_ref.shape[1]
    g = g_ref[0].reshape(K * t, 640)
    q = q_ref[0]
    xz = xz_ref[0]
    k_all = g[:, 0:256]
    v_all = g[:, 256:512]
    nx_all = g[:, 512:528]
    q_all = jnp.broadcast_to(q[None], (K, t, 256)).reshape(K * t, 256)
    xz_all = jnp.broadcast_to(xz[None], (K, t, 16)).reshape(K * t, 16)
    rel = xz_all - nx_all
    h = jnp.maximum(rel @ fdw1[...] + fdb1[...], 0.0)
    pos = h @ fdw2[...] + fdb2[...]
    gj = jnp.maximum((q_all - k_all + pos) @ fgw1[...] + fgb1[...], 0.0)
    a_all = (gj @ fgw2[...] + fgb2[...]) * jnp.float32(1.0 / 16.0)
    vp_all = v_all + pos
    m = a_all[0:t]
    for j in range(1, K):
        m = jnp.maximum(m, a_all[j * t:(j + 1) * t])
    e_all = jnp.exp(a_all - jnp.broadcast_to(
        m[None], (K, t, 256)).reshape(K * t, 256))
    den = e_all[0:t]
    num = e_all[0:t] * vp_all[0:t]
    for j in range(1, K):
        ej = e_all[j * t:(j + 1) * t]
        den = den + ej
        num = num + ej * vp_all[j * t:(j + 1) * t]
    res = num / den
    o_ref[0] = res @ fc2w[...] + fc2b[...] + pre_ref[0]


def _td_body(K, d, g_ref, nx_ref, w1a, w1b, b1, g1, be1, w2, b2, g2, be2,
             o_ref):
    sq = np.float32(np.sqrt(1.0 + 1e-05))
    nx = nx_ref[0]
    acc = None
    for j in range(K):
        gp = g_ref[0, j, :, 0:d]
        gx = g_ref[0, j, :, d:d + 16]
        norm = gx - nx
        y = norm @ w1a[...] + gp @ w1b[...] + b1[...]
        y = jnp.maximum((y / sq) * g1[...] + be1[...], 0.0)
        y = y @ w2[...] + b2[...]
        y = jnp.maximum((y / sq) * g2[...] + be2[...], 0.0)
        acc = y if acc is None else jnp.maximum(acc, y)
    o_ref[0] = acc


def _full(shape):
    nd = len(shape)
    return pl.BlockSpec(shape, lambda *_: (0,) * nd)


def _input_mlp(xf, w1, b1, w2, b2):
    R = xf.shape[0]
    t = min(R, 1024)
    return pl.pallas_call(
        _mlp_in_body,
        grid=(R // t,),
        in_specs=[
            pl.BlockSpec((t, 3), lambda i: (i, 0)),
            _full(w1.shape), _full(b1.shape), _full(w2.shape), _full(b2.shape),
        ],
        out_specs=pl.BlockSpec((t, 32), lambda i: (i, 0)),
        out_shape=jax.ShapeDtypeStruct((R, 32), jnp.float32),
        interpret=_INTERPRET,
    )(xf, w1, b1, w2, b2)


def _prep(featsf, xz16f, p):
    R, d = featsf.shape
    t = min(R, 512)
    fc1b = p['fc1_b'].reshape(1, -1)
    q, tab = pl.pallas_call(
        _prep_body,
        grid=(R // t,),
        in_specs=[
            pl.BlockSpec((t, d), lambda i: (i, 0)),
            pl.BlockSpec((t, 16), lambda i: (i, 0)),
            _full((d, 256)), _full((1, 256)),
            _full((256, 256)), _full((256, 256)), _full((256, 256)),
        ],
        out_specs=[
            pl.BlockSpec((t, 256), lambda i: (i, 0)),
            pl.BlockSpec((t, 640), lambda i: (i, 0)),
        ],
        out_shape=[
            jax.ShapeDtypeStruct((R, 256), jnp.float32),
            jax.ShapeDtypeStruct((R, 640), jnp.float32),
        ],
        interpret=_INTERPRET,
    )(featsf, xz16f, p['fc1_w'], fc1b, p['wq'], p['wk'], p['wv'])
    return q, tab


def _knn(q16, pT16, K):
    B, M, _ = q16.shape
    N = pT16.shape[2]
    t = min(M, 256)
    return pl.pallas_call(
        functools.partial(_knn_body, K, N),
        grid=(B, M // t),
        in_specs=[
            pl.BlockSpec((1, t, 16), lambda b, i: (b, i, 0)),
            pl.BlockSpec((1, 16, N), lambda b, i: (b, 0, 0)),
        ],
        out_specs=pl.BlockSpec((1, t, K), lambda b, i: (b, i, 0)),
        out_shape=jax.ShapeDtypeStruct((B, M, K), jnp.int32),
        interpret=_INTERPRET,
    )(q16, pT16)


def _fps(xz16, xzT16, M):
    B, N, _ = xz16.shape
    return pl.pallas_call(
        functools.partial(_fps_body, B, M, N),
        grid=(1,),
        in_specs=[
            pl.BlockSpec((B, N, 16), lambda i: (0, 0, 0)),
            pl.BlockSpec((B, 16, N), lambda i: (0, 0, 0)),
        ],
        out_specs=pl.BlockSpec((B, M, 16), lambda i: (0, 0, 0)),
        out_shape=jax.ShapeDtypeStruct((B, M, 16), jnp.float32),
        scratch_shapes=[pltpu.VMEM((B, N), jnp.float32),
                        pltpu.VMEM((B, 1), jnp.int32)],
        interpret=_INTERPRET,
    )(xz16, xzT16)


def _attn(gath, q, xz16, pre, p, K):
    B, M, d = pre.shape
    t = min(M, 128)
    fdw1 = jnp.concatenate(
        [p['fd_w1'], jnp.zeros((13, 256), jnp.float32)], axis=0)
    args = [gath, q, xz16, pre, fdw1,
            p['fd_b1'].reshape(1, -1), p['fd_w2'], p['fd_b2'].reshape(1, -1),
            p['fg_w1'], p['fg_b1'].reshape(1, -1),
            p['fg_w2'], p['fg_b2'].reshape(1, -1),
            p['fc2_w'], p['fc2_b'].reshape(1, -1)]
    return pl.pallas_call(
        functools.partial(_attn_body, K),
        grid=(B, M // t),
        in_specs=[
            pl.BlockSpec((1, K, t, 640), lambda b, i: (b, 0, i, 0)),
            pl.BlockSpec((1, t, 256), lambda b, i: (b, i, 0)),
            pl.BlockSpec((1, t, 16), lambda b, i: (b, i, 0)),
            pl.BlockSpec((1, t, d), lambda b, i: (b, i, 0)),
            _full((16, 256)), _full((1, 256)),
            _full((256, 256)), _full((1, 256)),
            _full((256, 256)), _full((1, 256)),
            _full((256, 256)), _full((1, 256)),
            _full((256, d)), _full((1, d)),
        ],
        out_specs=pl.BlockSpec((1, t, d), lambda b, i: (b, i, 0)),
        out_shape=jax.ShapeDtypeStruct((B, M, d), jnp.float32),
        interpret=_INTERPRET,
    )(*args)


def _td_conv(gath, nx16, p, d):
    B, K, M, D = gath.shape
    ch = p['convs'][0][0].shape[1]
    t = min(M, 128)
    (w1, b1, g1, be1), (w2, b2, g2, be2) = p['convs']
    w1a = jnp.concatenate(
        [w1[:3], jnp.zeros((13, ch), jnp.float32)], axis=0)
    w1b = w1[3:]
    args = [gath, nx16, w1a, w1b, b1.reshape(1, -1), g1.reshape(1, -1),
            be1.reshape(1, -1), w2, b2.reshape(1, -1), g2.reshape(1, -1),
            be2.reshape(1, -1)]
    return pl.pallas_call(
        functools.partial(_td_body, K, d),
        grid=(B, M // t),
        in_specs=[
            pl.BlockSpec((1, K, t, D), lambda b, i: (b, 0, i, 0)),
            pl.BlockSpec((1, t, 16), lambda b, i: (b, i, 0)),
            _full((16, ch)), _full((d, ch)), _full((1, ch)), _full((1, ch)),
            _full((1, ch)), _full((ch, ch)), _full((1, ch)), _full((1, ch)),
            _full((1, ch)),
        ],
        out_specs=pl.BlockSpec((1, t, ch), lambda b, i: (b, i, 0)),
        out_shape=jax.ShapeDtypeStruct((B, M, ch), jnp.float32),
        interpret=_INTERPRET,
    )(*args)


def _flat_idx(idx, N):
    B = idx.shape[0]
    off = (jnp.arange(B, dtype=jnp.int32) * N)[:, None, None]
    return (jnp.transpose(idx, (0, 2, 1)) + off).reshape(-1)


def _transformer(xz16, xzT16, feats, p, K):
    B, M, d = feats.shape
    idx = _knn(xz16, xzT16, K)
    q, tab = _prep(feats.reshape(B * M, d), xz16.reshape(B * M, 16), p)
    gath = _gather(tab, _flat_idx(idx, M)).reshape(B, K, M, 640)
    return _attn(gath, q.reshape(B, M, 256), xz16, feats, p, K)


def _transition(xz16, xzT16, points, M, p):
    B, N, d = points.shape
    new16 = _fps(xz16, xzT16, M)
    idx = _knn(new16, xzT16, 16)
    dpad = -(-(d + 16) // 128) * 128
    tab = jnp.concatenate(
        [points.reshape(B * N, d), xz16.reshape(B * N, 16),
         jnp.zeros((B * N, dpad - d - 16), jnp.float32)], axis=1)
    gath = _gather(tab, _flat_idx(idx, N)).reshape(B, 16, M, dpad)
    y = _td_conv(gath, new16, p, d)
    return new16, y


def kernel(x, params):
    B, N0, _ = x.shape
    xz16 = jnp.pad(x[..., :3], ((0, 0), (0, 0), (0, 13)))
    xzT16 = jnp.swapaxes(xz16, 1, 2)
    h = _input_mlp(x.reshape(B * N0, 3), params['in_w1'],
                   params['in_b1'].reshape(1, -1), params['in_w2'],
                   params['in_b2'].reshape(1, -1)).reshape(B, N0, 32)
    pts = _transformer(xz16, xzT16, h, params['t0'], 16)
    for i in range(4):
        M = N0 // 4 ** (i + 1)
        xz16, pts = _transition(xz16, xzT16, pts, M, params['td' + str(i)])
        xzT16 = jnp.swapaxes(xz16, 1, 2)
        pts = _transformer(xz16, xzT16, pts, params['t' + str(i + 1)],
                           min(16, M))
    return pts


# speedup vs baseline: 1.6390x; 1.0352x over previous
import functools

import jax
import jax.numpy as jnp
import numpy as np
from jax import lax
from jax.experimental import pallas as pl
from jax.experimental.pallas import tpu as pltpu
from jax.experimental.pallas import tpu_sc as plsc

_INTERPRET = False


def _sc_gather(table, idx):
    G = idx.shape[0]
    D = table.shape[1]
    info = plsc.get_sparse_core_info()
    nc, ns = info.num_cores, info.num_subcores
    nw = nc * ns
    assert G % nw == 0, (G, nw)
    per_w = G // nw
    chunk = 8
    while chunk * 2 <= per_w and chunk * 2 <= 128 and (chunk * 4) * (D + 1) <= 110000:
        chunk *= 2
    chunk = min(chunk, per_w)
    nch = per_w // chunk
    mesh = plsc.VectorSubcoreMesh(core_axis_name="c", subcore_axis_name="s")

    @functools.partial(
        pl.kernel,
        mesh=mesh,
        out_type=jax.ShapeDtypeStruct((G, D), jnp.float32),
        scratch_types=[
            pltpu.VMEM((chunk,), jnp.int32),
            pltpu.VMEM((chunk,), jnp.int32),
            pltpu.VMEM((chunk, D), jnp.float32),
            pltpu.VMEM((chunk, D), jnp.float32),
            pltpu.SemaphoreType.DMA,
            pltpu.SemaphoreType.DMA,
            pltpu.SemaphoreType.DMA,
            pltpu.SemaphoreType.DMA,
            pltpu.SemaphoreType.DMA,
        ],
    )
    def gk(tab_hbm, idx_hbm, out_hbm, idx0, idx1, rows0, rows1,
           isem0, isem1, gsem, osem0, osem1):
        idx_v = [idx0, idx1]
        rows_v = [rows0, rows1]
        isem = [isem0, isem1]
        osem = [osem0, osem1]
        wid = lax.axis_index("s") * nc + lax.axis_index("c")
        base = wid * per_w
        h_i = [None, None]
        h_o = [None, None]
        h_i[0] = pltpu.async_copy(
            idx_hbm.at[pl.ds(base, chunk)], idx_v[0], isem[0])
        for ci in range(nch):
            cur = ci % 2
            if ci + 1 < nch:
                off_n = base + (ci + 1) * chunk
                h_i[1 - cur] = pltpu.async_copy(
                    idx_hbm.at[pl.ds(off_n, chunk)], idx_v[1 - cur],
                    isem[1 - cur])
            h_i[cur].wait()
            if h_o[cur] is not None:
                h_o[cur].wait()
            pltpu.async_copy(tab_hbm.at[idx_v[cur]], rows_v[cur], gsem).wait()
            off = base + ci * chunk
            h_o[cur] = pltpu.async_copy(
                rows_v[cur], out_hbm.at[pl.ds(off, chunk)], osem[cur])
        for h in h_o:
            if h is not None:
                h.wait()

    return gk(table, idx)


_gather = _sc_gather


def _mlp_in_body(x_ref, w1, b1, w2, b2, o_ref):
    h = jnp.maximum(x_ref[...] @ w1[...] + b1[...], 0.0)
    o_ref[...] = h @ w2[...] + b2[...]


def _prep_body(f_ref, xz_ref, fc1w, fc1b, wq, wk, wv, q_ref, tab_ref):
    xx = f_ref[...] @ fc1w[...] + fc1b[...]
    q_ref[...] = xx @ wq[...]
    tab_ref[:, 0:256] = xx @ wk[...]
    tab_ref[:, 256:512] = xx @ wv[...]
    tab_ref[:, 512:528] = xz_ref[...]
    tab_ref[:, 528:640] = jnp.zeros_like(tab_ref[:, 528:640])


def _knn_body(K, N, q_ref, p_ref, o_ref):
    q = q_ref[0]
    p = p_ref[0]
    qq = jnp.sum(q * q, axis=1, keepdims=True)
    pp = jnp.sum(p * p, axis=0, keepdims=True)
    d = qq + pp - 2.0 * (q @ p)
    iota = lax.broadcasted_iota(jnp.int32, d.shape, 1)
    cols = []
    for _ in range(K):
        m = jnp.min(d, axis=1, keepdims=True)
        sel = jnp.min(jnp.where(d == m, iota, N), axis=1, keepdims=True)
        cols.append(sel)
        d = jnp.where(iota == sel, jnp.float32(np.inf), d)
    o_ref[0] = jnp.concatenate(cols, axis=1)


def _fps_body(B, M, N, xr_ref, xt_ref, o_ref, dist_ref, sel_ref):
    iota = lax.broadcasted_iota(jnp.int32, (B, N), 1)
    dist_ref[...] = jnp.full((B, N), 1e10, jnp.float32)
    sel_ref[...] = jnp.zeros((B, 1), jnp.int32)

    def body(i, carry):
        rows = []
        for b in range(B):
            far = sel_ref[b, 0]
            o_ref[b, pl.ds(i, 1), :] = xr_ref[b, pl.ds(far, 1), :]
            cx = xr_ref[b, far, 0]
            cy = xr_ref[b, far, 1]
            cz = xr_ref[b, far, 2]
            s0 = xt_ref[b, 0:1, :] - cx
            s1 = xt_ref[b, 1:2, :] - cy
            s2 = xt_ref[b, 2:3, :] - cz
            d2 = s0 * s0 + s1 * s1
            d2 = d2 + s2 * s2
            rows.append(d2)
        dist = jnp.minimum(dist_ref[...], jnp.concatenate(rows, axis=0))
        dist_ref[...] = dist
        m = jnp.max(dist, axis=1, keepdims=True)
        sel_ref[...] = jnp.min(jnp.where(dist == m, iota, N), axis=1,
                               keepdims=True).astype(jnp.int32)
        return carry

    lax.fori_loop(0, M, body, jnp.int32(0))


def _attn_body(K, g_ref, q_ref, xz_ref, pre_ref, fdw1, fdb1, fdw2, fdb2,
               fgw1, fgb1, fgw2, fgb2, fc2w, fc2b, o_ref):
    t = q_ref.shape[1]
    g = g_ref[0].reshape(K * t, 640)
    q = q_ref[0]
    xz = xz_ref[0]
    k_all = g[:, 0:256]
    v_all = g[:, 256:512]
    nx_all = g[:, 512:528]
    q_all = jnp.broadcast_to(q[None], (K, t, 256)).reshape(K * t, 256)
    xz_all = jnp.broadcast_to(xz[None], (K, t, 16)).reshape(K * t, 16)
    rel = xz_all - nx_all
    h = jnp.maximum(rel @ fdw1[...] + fdb1[...], 0.0)
    pos = h @ fdw2[...] + fdb2[...]
    gj = jnp.maximum((q_all - k_all + pos) @ fgw1[...] + fgb1[...], 0.0)
    a_all = (gj @ fgw2[...] + fgb2[...]) * jnp.float32(1.0 / 16.0)
    vp_all = v_all + pos
    m = a_all[0:t]
    for j in range(1, K):
        m = jnp.maximum(m, a_all[j * t:(j + 1) * t])
    e_all = jnp.exp(a_all - jnp.broadcast_to(
        m[None], (K, t, 256)).reshape(K * t, 256))
    den = e_all[0:t]
    num = e_all[0:t] * vp_all[0:t]
    for j in range(1, K):
        ej = e_all[j * t:(j + 1) * t]
        den = den + ej
        num = num + ej * vp_all[j * t:(j + 1) * t]
    res = num / den
    o_ref[0] = res @ fc2w[...] + fc2b[...] + pre_ref[0]


def _td_body(K, d, g_ref, nx_ref, w1a, w1b, b1, g1, be1, w2, b2, g2, be2,
             o_ref):
    sq = np.float32(np.sqrt(1.0 + 1e-05))
    t = nx_ref.shape[1]
    D = g_ref.shape[3]
    g = g_ref[0].reshape(K * t, D)
    nx = nx_ref[0]
    gp = g[:, 0:d]
    gx = g[:, d:d + 16]
    norm = gx - jnp.broadcast_to(nx[None], (K, t, 16)).reshape(K * t, 16)
    y = norm @ w1a[...] + gp @ w1b[...] + b1[...]
    y = jnp.maximum((y / sq) * g1[...] + be1[...], 0.0)
    y = y @ w2[...] + b2[...]
    y = jnp.maximum((y / sq) * g2[...] + be2[...], 0.0)
    acc = y[0:t]
    for j in range(1, K):
        acc = jnp.maximum(acc, y[j * t:(j + 1) * t])
    o_ref[0] = acc


def _full(shape):
    nd = len(shape)
    return pl.BlockSpec(shape, lambda *_: (0,) * nd)


def _input_mlp(xf, w1, b1, w2, b2):
    R = xf.shape[0]
    t = min(R, 1024)
    return pl.pallas_call(
        _mlp_in_body,
        grid=(R // t,),
        in_specs=[
            pl.BlockSpec((t, 3), lambda i: (i, 0)),
            _full(w1.shape), _full(b1.shape), _full(w2.shape), _full(b2.shape),
        ],
        out_specs=pl.BlockSpec((t, 32), lambda i: (i, 0)),
        out_shape=jax.ShapeDtypeStruct((R, 32), jnp.float32),
        interpret=_INTERPRET,
    )(xf, w1, b1, w2, b2)


def _prep(featsf, xz16f, p):
    R, d = featsf.shape
    t = min(R, 512)
    fc1b = p['fc1_b'].reshape(1, -1)
    q, tab = pl.pallas_call(
        _prep_body,
        grid=(R // t,),
        in_specs=[
            pl.BlockSpec((t, d), lambda i: (i, 0)),
            pl.BlockSpec((t, 16), lambda i: (i, 0)),
            _full((d, 256)), _full((1, 256)),
            _full((256, 256)), _full((256, 256)), _full((256, 256)),
        ],
        out_specs=[
            pl.BlockSpec((t, 256), lambda i: (i, 0)),
            pl.BlockSpec((t, 640), lambda i: (i, 0)),
        ],
        out_shape=[
            jax.ShapeDtypeStruct((R, 256), jnp.float32),
            jax.ShapeDtypeStruct((R, 640), jnp.float32),
        ],
        interpret=_INTERPRET,
    )(featsf, xz16f, p['fc1_w'], fc1b, p['wq'], p['wk'], p['wv'])
    return q, tab


def _knn(q16, pT16, K):
    B, M, _ = q16.shape
    N = pT16.shape[2]
    t = min(M, 256)
    return pl.pallas_call(
        functools.partial(_knn_body, K, N),
        grid=(B, M // t),
        in_specs=[
            pl.BlockSpec((1, t, 16), lambda b, i: (b, i, 0)),
            pl.BlockSpec((1, 16, N), lambda b, i: (b, 0, 0)),
        ],
        out_specs=pl.BlockSpec((1, t, K), lambda b, i: (b, i, 0)),
        out_shape=jax.ShapeDtypeStruct((B, M, K), jnp.int32),
        interpret=_INTERPRET,
    )(q16, pT16)


def _fps(xz16, xzT16, M):
    B, N, _ = xz16.shape
    return pl.pallas_call(
        functools.partial(_fps_body, B, M, N),
        grid=(1,),
        in_specs=[
            pl.BlockSpec((B, N, 16), lambda i: (0, 0, 0)),
            pl.BlockSpec((B, 16, N), lambda i: (0, 0, 0)),
        ],
        out_specs=pl.BlockSpec((B, M, 16), lambda i: (0, 0, 0)),
        out_shape=jax.ShapeDtypeStruct((B, M, 16), jnp.float32),
        scratch_shapes=[pltpu.VMEM((B, N), jnp.float32),
                        pltpu.VMEM((B, 1), jnp.int32)],
        interpret=_INTERPRET,
    )(xz16, xzT16)


def _attn(gath, q, xz16, pre, p, K):
    B, M, d = pre.shape
    t = min(M, 128)
    fdw1 = jnp.concatenate(
        [p['fd_w1'], jnp.zeros((13, 256), jnp.float32)], axis=0)
    args = [gath, q, xz16, pre, fdw1,
            p['fd_b1'].reshape(1, -1), p['fd_w2'], p['fd_b2'].reshape(1, -1),
            p['fg_w1'], p['fg_b1'].reshape(1, -1),
            p['fg_w2'], p['fg_b2'].reshape(1, -1),
            p['fc2_w'], p['fc2_b'].reshape(1, -1)]
    return pl.pallas_call(
        functools.partial(_attn_body, K),
        grid=(B, M // t),
        in_specs=[
            pl.BlockSpec((1, K, t, 640), lambda b, i: (b, 0, i, 0)),
            pl.BlockSpec((1, t, 256), lambda b, i: (b, i, 0)),
            pl.BlockSpec((1, t, 16), lambda b, i: (b, i, 0)),
            pl.BlockSpec((1, t, d), lambda b, i: (b, i, 0)),
            _full((16, 256)), _full((1, 256)),
            _full((256, 256)), _full((1, 256)),
            _full((256, 256)), _full((1, 256)),
            _full((256, 256)), _full((1, 256)),
            _full((256, d)), _full((1, d)),
        ],
        out_specs=pl.BlockSpec((1, t, d), lambda b, i: (b, i, 0)),
        out_shape=jax.ShapeDtypeStruct((B, M, d), jnp.float32),
        interpret=_INTERPRET,
    )(*args)


def _td_conv(gath, nx16, p, d):
    B, K, M, D = gath.shape
    ch = p['convs'][0][0].shape[1]
    t = min(M, 128)
    (w1, b1, g1, be1), (w2, b2, g2, be2) = p['convs']
    w1a = jnp.concatenate(
        [w1[:3], jnp.zeros((13, ch), jnp.float32)], axis=0)
    w1b = w1[3:]
    args = [gath, nx16, w1a, w1b, b1.reshape(1, -1), g1.reshape(1, -1),
            be1.reshape(1, -1), w2, b2.reshape(1, -1), g2.reshape(1, -1),
            be2.reshape(1, -1)]
    return pl.pallas_call(
        functools.partial(_td_body, K, d),
        grid=(B, M // t),
        in_specs=[
            pl.BlockSpec((1, K, t, D), lambda b, i: (b, 0, i, 0)),
            pl.BlockSpec((1, t, 16), lambda b, i: (b, i, 0)),
            _full((16, ch)), _full((d, ch)), _full((1, ch)), _full((1, ch)),
            _full((1, ch)), _full((ch, ch)), _full((1, ch)), _full((1, ch)),
            _full((1, ch)),
        ],
        out_specs=pl.BlockSpec((1, t, ch), lambda b, i: (b, i, 0)),
        out_shape=jax.ShapeDtypeStruct((B, M, ch), jnp.float32),
        interpret=_INTERPRET,
    )(*args)


def _flat_idx(idx, N):
    B = idx.shape[0]
    off = (jnp.arange(B, dtype=jnp.int32) * N)[:, None, None]
    return (jnp.transpose(idx, (0, 2, 1)) + off).reshape(-1)


def _transformer(xz16, xzT16, feats, p, K):
    B, M, d = feats.shape
    idx = _knn(xz16, xzT16, K)
    q, tab = _prep(feats.reshape(B * M, d), xz16.reshape(B * M, 16), p)
    gath = _gather(tab, _flat_idx(idx, M)).reshape(B, K, M, 640)
    return _attn(gath, q.reshape(B, M, 256), xz16, feats, p, K)


def _transition(xz16, xzT16, points, M, p):
    B, N, d = points.shape
    new16 = _fps(xz16, xzT16, M)
    idx = _knn(new16, xzT16, 16)
    dpad = -(-(d + 16) // 128) * 128
    tab = jnp.concatenate(
        [points.reshape(B * N, d), xz16.reshape(B * N, 16),
         jnp.zeros((B * N, dpad - d - 16), jnp.float32)], axis=1)
    gath = _gather(tab, _flat_idx(idx, N)).reshape(B, 16, M, dpad)
    y = _td_conv(gath, new16, p, d)
    return new16, y


def kernel(x, params):
    B, N0, _ = x.shape
    xz16 = jnp.pad(x[..., :3], ((0, 0), (0, 0), (0, 13)))
    xzT16 = jnp.swapaxes(xz16, 1, 2)
    h = _input_mlp(x.reshape(B * N0, 3), params['in_w1'],
                   params['in_b1'].reshape(1, -1), params['in_w2'],
                   params['in_b2'].reshape(1, -1)).reshape(B, N0, 32)
    pts = _transformer(xz16, xzT16, h, params['t0'], 16)
    for i in range(4):
        M = N0 // 4 ** (i + 1)
        xz16, pts = _transition(xz16, xzT16, pts, M, params['td' + str(i)])
        xzT16 = jnp.swapaxes(xz16, 1, 2)
        pts = _transformer(xz16, xzT16, pts, params['t' + str(i + 1)],
                           min(16, M))
    return pts


# speedup vs baseline: 1.6648x; 1.0158x over previous
import functools

import jax
import jax.numpy as jnp
import numpy as np
from jax import lax
from jax.experimental import pallas as pl
from jax.experimental.pallas import tpu as pltpu
from jax.experimental.pallas import tpu_sc as plsc

_INTERPRET = False


def _sc_gather(table, idx):
    G = idx.shape[0]
    D = table.shape[1]
    info = plsc.get_sparse_core_info()
    nc, ns = info.num_cores, info.num_subcores
    nw = nc * ns
    assert G % nw == 0, (G, nw)
    per_w = G // nw
    chunk = 8
    while chunk * 2 <= per_w and chunk * 2 <= 128 and (chunk * 4) * (D + 1) <= 110000:
        chunk *= 2
    chunk = min(chunk, per_w)
    nch = per_w // chunk
    mesh = plsc.VectorSubcoreMesh(core_axis_name="c", subcore_axis_name="s")

    @functools.partial(
        pl.kernel,
        mesh=mesh,
        out_type=jax.ShapeDtypeStruct((G, D), jnp.float32),
        scratch_types=[
            pltpu.VMEM((chunk,), jnp.int32),
            pltpu.VMEM((chunk,), jnp.int32),
            pltpu.VMEM((chunk, D), jnp.float32),
            pltpu.VMEM((chunk, D), jnp.float32),
            pltpu.SemaphoreType.DMA,
            pltpu.SemaphoreType.DMA,
            pltpu.SemaphoreType.DMA,
            pltpu.SemaphoreType.DMA,
            pltpu.SemaphoreType.DMA,
        ],
    )
    def gk(tab_hbm, idx_hbm, out_hbm, idx0, idx1, rows0, rows1,
           isem0, isem1, gsem, osem0, osem1):
        idx_v = [idx0, idx1]
        rows_v = [rows0, rows1]
        isem = [isem0, isem1]
        osem = [osem0, osem1]
        wid = lax.axis_index("s") * nc + lax.axis_index("c")
        base = wid * per_w
        h_i = [None, None]
        h_o = [None, None]
        h_i[0] = pltpu.async_copy(
            idx_hbm.at[pl.ds(base, chunk)], idx_v[0], isem[0])
        for ci in range(nch):
            cur = ci % 2
            if ci + 1 < nch:
                off_n = base + (ci + 1) * chunk
                h_i[1 - cur] = pltpu.async_copy(
                    idx_hbm.at[pl.ds(off_n, chunk)], idx_v[1 - cur],
                    isem[1 - cur])
            h_i[cur].wait()
            if h_o[cur] is not None:
                h_o[cur].wait()
            pltpu.async_copy(tab_hbm.at[idx_v[cur]], rows_v[cur], gsem).wait()
            off = base + ci * chunk
            h_o[cur] = pltpu.async_copy(
                rows_v[cur], out_hbm.at[pl.ds(off, chunk)], osem[cur])
        for h in h_o:
            if h is not None:
                h.wait()

    return gk(table, idx)


_gather = _sc_gather


def _mlp_in_body(x_ref, w1, b1, w2, b2, o_ref):
    h = jnp.maximum(x_ref[...] @ w1[...] + b1[...], 0.0)
    o_ref[...] = h @ w2[...] + b2[...]


def _prep_body(f_ref, xz_ref, fc1w, fc1b, wq, wk, wv, q_ref, tab_ref):
    xx = f_ref[...] @ fc1w[...] + fc1b[...]
    q_ref[...] = xx @ wq[...]
    tab_ref[:, 0:256] = xx @ wk[...]
    tab_ref[:, 256:512] = xx @ wv[...]
    tab_ref[:, 512:528] = xz_ref[...]
    tab_ref[:, 528:640] = jnp.zeros_like(tab_ref[:, 528:640])


def _knn_body(K, N, q_ref, p_ref, o_ref):
    q = q_ref[0]
    p = p_ref[0]
    qq = jnp.sum(q * q, axis=1, keepdims=True)
    pp = jnp.sum(p * p, axis=0, keepdims=True)
    d = qq + pp - 2.0 * (q @ p)
    iota = lax.broadcasted_iota(jnp.int32, d.shape, 1)
    cols = []
    for _ in range(K):
        m = jnp.min(d, axis=1, keepdims=True)
        sel = jnp.min(jnp.where(d == m, iota, N), axis=1, keepdims=True)
        cols.append(sel)
        d = jnp.where(iota == sel, jnp.float32(np.inf), d)
    o_ref[0] = jnp.concatenate(cols, axis=1)


def _fps_body(B, M, N, xr_ref, xt_ref, o_ref, dist_ref, sel_ref):
    iota = lax.broadcasted_iota(jnp.int32, (B, N), 1)
    dist_ref[...] = jnp.full((B, N), 1e10, jnp.float32)
    sel_ref[...] = jnp.zeros((B, 1), jnp.int32)

    def body(i, carry):
        rows = []
        for b in range(B):
            far = sel_ref[b, 0]
            o_ref[b, pl.ds(i, 1), :] = xr_ref[b, pl.ds(far, 1), :]
            cx = xr_ref[b, far, 0]
            cy = xr_ref[b, far, 1]
            cz = xr_ref[b, far, 2]
            s0 = xt_ref[b, 0:1, :] - cx
            s1 = xt_ref[b, 1:2, :] - cy
            s2 = xt_ref[b, 2:3, :] - cz
            d2 = s0 * s0 + s1 * s1
            d2 = d2 + s2 * s2
            rows.append(d2)
        dist = jnp.minimum(dist_ref[...], jnp.concatenate(rows, axis=0))
        dist_ref[...] = dist
        m = jnp.max(dist, axis=1, keepdims=True)
        sel_ref[...] = jnp.min(jnp.where(dist == m, iota, N), axis=1,
                               keepdims=True).astype(jnp.int32)
        return carry

    lax.fori_loop(0, M, body, jnp.int32(0))


def _attn_body(K, g_ref, q_ref, xz_ref, pre_ref, fdw1, fdb1, fdw2, fdb2,
               fgw1, fgb1, fgw2, fgb2, fc2w, fc2b, o_ref):
    t = q_ref.shape[1]
    g = g_ref[0].reshape(K * t, 640)
    q = q_ref[0]
    xz = xz_ref[0]
    k_all = g[:, 0:256]
    v_all = g[:, 256:512]
    nx_all = g[:, 512:528]
    q_all = jnp.broadcast_to(q[None], (K, t, 256)).reshape(K * t, 256)
    xz_all = jnp.broadcast_to(xz[None], (K, t, 16)).reshape(K * t, 16)
    rel = xz_all - nx_all
    h = jnp.maximum(rel @ fdw1[...] + fdb1[...], 0.0)
    pos = h @ fdw2[...] + fdb2[...]
    gj = jnp.maximum((q_all - k_all + pos) @ fgw1[...] + fgb1[...], 0.0)
    a_all = (gj @ fgw2[...] + fgb2[...]) * jnp.float32(1.0 / 16.0)
    vp_all = v_all + pos
    m = a_all[0:t]
    for j in range(1, K):
        m = jnp.maximum(m, a_all[j * t:(j + 1) * t])
    e_all = jnp.exp(a_all - jnp.broadcast_to(
        m[None], (K, t, 256)).reshape(K * t, 256))
    den = e_all[0:t]
    num = e_all[0:t] * vp_all[0:t]
    for j in range(1, K):
        ej = e_all[j * t:(j + 1) * t]
        den = den + ej
        num = num + ej * vp_all[j * t:(j + 1) * t]
    res = num / den
    o_ref[0] = res @ fc2w[...] + fc2b[...] + pre_ref[0]


def _td_body(K, d, g_ref, nx_ref, w1a, w1b, b1, g1, be1, w2, b2, g2, be2,
             o_ref):
    sq = np.float32(np.sqrt(1.0 + 1e-05))
    t = nx_ref.shape[1]
    D = g_ref.shape[3]
    g = g_ref[0].reshape(K * t, D)
    nx = nx_ref[0]
    gp = g[:, 0:d]
    gx = g[:, d:d + 16]
    norm = gx - jnp.broadcast_to(nx[None], (K, t, 16)).reshape(K * t, 16)
    y = norm @ w1a[...] + gp @ w1b[...] + b1[...]
    y = jnp.maximum((y / sq) * g1[...] + be1[...], 0.0)
    y = y @ w2[...] + b2[...]
    y = jnp.maximum((y / sq) * g2[...] + be2[...], 0.0)
    acc = y[0:t]
    for j in range(1, K):
        acc = jnp.maximum(acc, y[j * t:(j + 1) * t])
    o_ref[0] = acc


def _full(shape):
    nd = len(shape)
    return pl.BlockSpec(shape, lambda *_: (0,) * nd)


def _input_mlp(xf, w1, b1, w2, b2):
    R = xf.shape[0]
    t = min(R, 1024)
    return pl.pallas_call(
        _mlp_in_body,
        grid=(R // t,),
        in_specs=[
            pl.BlockSpec((t, 3), lambda i: (i, 0)),
            _full(w1.shape), _full(b1.shape), _full(w2.shape), _full(b2.shape),
        ],
        out_specs=pl.BlockSpec((t, 32), lambda i: (i, 0)),
        out_shape=jax.ShapeDtypeStruct((R, 32), jnp.float32),
        interpret=_INTERPRET,
    )(xf, w1, b1, w2, b2)


def _prep(featsf, xz16f, p):
    R, d = featsf.shape
    t = min(R, 512)
    fc1b = p['fc1_b'].reshape(1, -1)
    q, tab = pl.pallas_call(
        _prep_body,
        grid=(R // t,),
        in_specs=[
            pl.BlockSpec((t, d), lambda i: (i, 0)),
            pl.BlockSpec((t, 16), lambda i: (i, 0)),
            _full((d, 256)), _full((1, 256)),
            _full((256, 256)), _full((256, 256)), _full((256, 256)),
        ],
        out_specs=[
            pl.BlockSpec((t, 256), lambda i: (i, 0)),
            pl.BlockSpec((t, 640), lambda i: (i, 0)),
        ],
        out_shape=[
            jax.ShapeDtypeStruct((R, 256), jnp.float32),
            jax.ShapeDtypeStruct((R, 640), jnp.float32),
        ],
        interpret=_INTERPRET,
    )(featsf, xz16f, p['fc1_w'], fc1b, p['wq'], p['wk'], p['wv'])
    return q, tab


def _knn(q16, pT16, K):
    B, M, _ = q16.shape
    N = pT16.shape[2]
    t = min(M, 256)
    return pl.pallas_call(
        functools.partial(_knn_body, K, N),
        grid=(B, M // t),
        in_specs=[
            pl.BlockSpec((1, t, 16), lambda b, i: (b, i, 0)),
            pl.BlockSpec((1, 16, N), lambda b, i: (b, 0, 0)),
        ],
        out_specs=pl.BlockSpec((1, t, K), lambda b, i: (b, i, 0)),
        out_shape=jax.ShapeDtypeStruct((B, M, K), jnp.int32),
        interpret=_INTERPRET,
    )(q16, pT16)


def _fps(xz16, xzT16, M):
    B, N, _ = xz16.shape
    return pl.pallas_call(
        functools.partial(_fps_body, B, M, N),
        grid=(1,),
        in_specs=[
            pl.BlockSpec((B, N, 16), lambda i: (0, 0, 0)),
            pl.BlockSpec((B, 16, N), lambda i: (0, 0, 0)),
        ],
        out_specs=pl.BlockSpec((B, M, 16), lambda i: (0, 0, 0)),
        out_shape=jax.ShapeDtypeStruct((B, M, 16), jnp.float32),
        scratch_shapes=[pltpu.VMEM((B, N), jnp.float32),
                        pltpu.VMEM((B, 1), jnp.int32)],
        interpret=_INTERPRET,
    )(xz16, xzT16)


def _attn(gath, q, xz16, pre, p, K):
    B, M, d = pre.shape
    t = min(M, 256)
    fdw1 = jnp.concatenate(
        [p['fd_w1'], jnp.zeros((13, 256), jnp.float32)], axis=0)
    args = [gath, q, xz16, pre, fdw1,
            p['fd_b1'].reshape(1, -1), p['fd_w2'], p['fd_b2'].reshape(1, -1),
            p['fg_w1'], p['fg_b1'].reshape(1, -1),
            p['fg_w2'], p['fg_b2'].reshape(1, -1),
            p['fc2_w'], p['fc2_b'].reshape(1, -1)]
    return pl.pallas_call(
        functools.partial(_attn_body, K),
        grid=(B, M // t),
        in_specs=[
            pl.BlockSpec((1, K, t, 640), lambda b, i: (b, 0, i, 0)),
            pl.BlockSpec((1, t, 256), lambda b, i: (b, i, 0)),
            pl.BlockSpec((1, t, 16), lambda b, i: (b, i, 0)),
            pl.BlockSpec((1, t, d), lambda b, i: (b, i, 0)),
            _full((16, 256)), _full((1, 256)),
            _full((256, 256)), _full((1, 256)),
            _full((256, 256)), _full((1, 256)),
            _full((256, 256)), _full((1, 256)),
            _full((256, d)), _full((1, d)),
        ],
        out_specs=pl.BlockSpec((1, t, d), lambda b, i: (b, i, 0)),
        out_shape=jax.ShapeDtypeStruct((B, M, d), jnp.float32),
        interpret=_INTERPRET,
    )(*args)


def _td_conv(gath, nx16, p, d):
    B, K, M, D = gath.shape
    ch = p['convs'][0][0].shape[1]
    t = min(M, 128)
    (w1, b1, g1, be1), (w2, b2, g2, be2) = p['convs']
    w1a = jnp.concatenate(
        [w1[:3], jnp.zeros((13, ch), jnp.float32)], axis=0)
    w1b = w1[3:]
    args = [gath, nx16, w1a, w1b, b1.reshape(1, -1), g1.reshape(1, -1),
            be1.reshape(1, -1), w2, b2.reshape(1, -1), g2.reshape(1, -1),
            be2.reshape(1, -1)]
    return pl.pallas_call(
        functools.partial(_td_body, K, d),
        grid=(B, M // t),
        in_specs=[
            pl.BlockSpec((1, K, t, D), lambda b, i: (b, 0, i, 0)),
            pl.BlockSpec((1, t, 16), lambda b, i: (b, i, 0)),
            _full((16, ch)), _full((d, ch)), _full((1, ch)), _full((1, ch)),
            _full((1, ch)), _full((ch, ch)), _full((1, ch)), _full((1, ch)),
            _full((1, ch)),
        ],
        out_specs=pl.BlockSpec((1, t, ch), lambda b, i: (b, i, 0)),
        out_shape=jax.ShapeDtypeStruct((B, M, ch), jnp.float32),
        interpret=_INTERPRET,
    )(*args)


def _flat_idx(idx, N):
    B = idx.shape[0]
    off = (jnp.arange(B, dtype=jnp.int32) * N)[:, None, None]
    return (jnp.transpose(idx, (0, 2, 1)) + off).reshape(-1)


def _transformer(xz16, xzT16, feats, p, K):
    B, M, d = feats.shape
    idx = _knn(xz16, xzT16, K)
    q, tab = _prep(feats.reshape(B * M, d), xz16.reshape(B * M, 16), p)
    gath = _gather(tab, _flat_idx(idx, M)).reshape(B, K, M, 640)
    return _attn(gath, q.reshape(B, M, 256), xz16, feats, p, K)


def _transition(xz16, xzT16, points, M, p):
    B, N, d = points.shape
    new16 = _fps(xz16, xzT16, M)
    idx = _knn(new16, xzT16, 16)
    dpad = -(-(d + 16) // 128) * 128
    tab = jnp.concatenate(
        [points.reshape(B * N, d), xz16.reshape(B * N, 16),
         jnp.zeros((B * N, dpad - d - 16), jnp.float32)], axis=1)
    gath = _gather(tab, _flat_idx(idx, N)).reshape(B, 16, M, dpad)
    y = _td_conv(gath, new16, p, d)
    return new16, y


def kernel(x, params):
    B, N0, _ = x.shape
    xz16 = jnp.pad(x[..., :3], ((0, 0), (0, 0), (0, 13)))
    xzT16 = jnp.swapaxes(xz16, 1, 2)
    h = _input_mlp(x.reshape(B * N0, 3), params['in_w1'],
                   params['in_b1'].reshape(1, -1), params['in_w2'],
                   params['in_b2'].reshape(1, -1)).reshape(B, N0, 32)
    pts = _transformer(xz16, xzT16, h, params['t0'], 16)
    for i in range(4):
        M = N0 // 4 ** (i + 1)
        xz16, pts = _transition(xz16, xzT16, pts, M, params['td' + str(i)])
        xzT16 = jnp.swapaxes(xz16, 1, 2)
        pts = _transformer(xz16, xzT16, pts, params['t' + str(i + 1)],
                           min(16, M))
    return pts
